# Initial kernel scaffold; baseline (speedup 1.0000x reference)
import jax
import jax.numpy as jnp
from jax.experimental import pallas as pl


def kernel(input, hidden_h, hidden_c, encoder_outputs, syn_embeddeds, edge_index, pg_mat, emb_table, W_attn, W_gcn, b_gcn, W_comb, b_comb, W_ih, W_hh, b_ih, b_hh, W_out, b_out, W_wh, W_ws, W_wx, b_wx):
    raise NotImplementedError("write your pallas kernel here")


# speedup vs baseline: 17.1796x; 17.1796x over previous
import functools

import jax
import jax.numpy as jnp
from jax import lax
from jax.experimental import pallas as pl
from jax.experimental.pallas import tpu as pltpu
from jax.experimental.pallas import tpu_sc as plsc

H = 256
HID4 = 4 * H

_NC = 2
_NS = 16
_NW = _NC * _NS


def _stage_a(input_i32, emb3, h0, c0, W_ih, W_hh, b_ih2, b_hh2, W_attn,
             W_ws, W_wx, b_wx2):
    def body(inp_ref, emb_ref, h_ref, c_ref, wih_ref, whh_ref, bih_ref,
             bhh_ref, wattn_ref, wws_ref, wwx_ref, bwx_ref,
             h2_ref, c2_ref, q_ref, phx_ref):
        x = emb_ref[0]
        h = h_ref[...]
        c = c_ref[...]
        g = (lax.dot_general(x, wih_ref[...], (((1,), (1,)), ((), ())),
                             preferred_element_type=jnp.float32)
             + lax.dot_general(h, whh_ref[...], (((1,), (1,)), ((), ())),
                               preferred_element_type=jnp.float32)
             + bih_ref[...] + bhh_ref[...])
        i_g = jax.nn.sigmoid(g[:, 0:H])
        f_g = jax.nn.sigmoid(g[:, H:2 * H])
        g_g = jnp.tanh(g[:, 2 * H:3 * H])
        o_g = jax.nn.sigmoid(g[:, 3 * H:4 * H])
        c2 = f_g * c + i_g * g_g
        h2 = o_g * jnp.tanh(c2)
        h2_ref[...] = h2
        c2_ref[...] = c2
        q_ref[...] = lax.dot_general(h2, wattn_ref[...],
                                     (((1,), (1,)), ((), ())),
                                     preferred_element_type=jnp.float32)
        phx = (jnp.sum(h2 * wws_ref[...]) + jnp.sum(x * wwx_ref[...])
               + bwx_ref[0, 0])
        phx_ref[...] = jnp.full((1, 1), 0.0, jnp.float32) + phx

    grid_spec = pltpu.PrefetchScalarGridSpec(
        num_scalar_prefetch=1,
        grid=(1,),
        in_specs=[
            pl.BlockSpec((1, 1, H), lambda i, inp: (inp[0], 0, 0)),
            pl.BlockSpec((1, H), lambda i, inp: (0, 0)),
            pl.BlockSpec((1, H), lambda i, inp: (0, 0)),
            pl.BlockSpec((HID4, H), lambda i, inp: (0, 0)),
            pl.BlockSpec((HID4, H), lambda i, inp: (0, 0)),
            pl.BlockSpec((1, HID4), lambda i, inp: (0, 0)),
            pl.BlockSpec((1, HID4), lambda i, inp: (0, 0)),
            pl.BlockSpec((H, H), lambda i, inp: (0, 0)),
            pl.BlockSpec((1, H), lambda i, inp: (0, 0)),
            pl.BlockSpec((1, H), lambda i, inp: (0, 0)),
            pl.BlockSpec((1, 1), lambda i, inp: (0, 0)),
        ],
        out_specs=[
            pl.BlockSpec((1, H), lambda i, inp: (0, 0)),
            pl.BlockSpec((1, H), lambda i, inp: (0, 0)),
            pl.BlockSpec((1, H), lambda i, inp: (0, 0)),
            pl.BlockSpec((1, 1), lambda i, inp: (0, 0)),
        ],
    )
    return pl.pallas_call(
        body,
        grid_spec=grid_spec,
        out_shape=[
            jax.ShapeDtypeStruct((1, H), jnp.float32),
            jax.ShapeDtypeStruct((1, H), jnp.float32),
            jax.ShapeDtypeStruct((1, H), jnp.float32),
            jax.ShapeDtypeStruct((1, 1), jnp.float32),
        ],
    )(input_i32, emb3, h0, c0, W_ih, W_hh, b_ih2, b_hh2, W_attn, W_ws,
      W_wx, b_wx2)


def _stage_b(q, syn, E, BE):
    steps = E // BE

    def body(q_ref, syn_ref, s_ref, m_ref, zi_ref, scr):
        i = pl.program_id(0)

        @pl.when(i == 0)
        def _():
            scr[0] = -jnp.inf
            scr[1] = 0.0

        s = lax.dot_general(q_ref[...], syn_ref[...],
                            (((1,), (1,)), ((), ())),
                            preferred_element_type=jnp.float32)
        s_ref[...] = s
        m_old = scr[0]
        z_old = scr[1]
        m_blk = jnp.max(s)
        m_new = jnp.maximum(m_old, m_blk)
        z_new = z_old * jnp.exp(m_old - m_new) + jnp.sum(jnp.exp(s - m_new))
        scr[0] = m_new
        scr[1] = z_new

        @pl.when(i == steps - 1)
        def _():
            m_ref[...] = jnp.full((1, 128), m_new, jnp.float32)
            zi_ref[...] = jnp.full((1, 128), 1.0 / z_new, jnp.float32)

    return pl.pallas_call(
        body,
        grid=(steps,),
        in_specs=[
            pl.BlockSpec((1, H), lambda i: (0, 0)),
            pl.BlockSpec((BE, H), lambda i: (i, 0)),
        ],
        out_specs=[
            pl.BlockSpec((1, BE), lambda i: (0, i)),
            pl.BlockSpec((1, 128), lambda i: (0, 0)),
            pl.BlockSpec((1, 128), lambda i: (0, 0)),
        ],
        out_shape=[
            jax.ShapeDtypeStruct((1, E), jnp.float32),
            jax.ShapeDtypeStruct((1, 128), jnp.float32),
            jax.ShapeDtypeStruct((1, 128), jnp.float32),
        ],
        scratch_shapes=[pltpu.SMEM((2,), jnp.float32)],
    )(q, syn)


def _stage_c(scores, mrow, zirow, pg_mat, E, EXT, BC):
    steps = E // BC

    def body(s_ref, m_ref, zi_ref, pg_ref, aw_ref, pv_ref):
        i = pl.program_id(0)
        aw = jnp.exp(s_ref[...] - m_ref[0, 0]) * zi_ref[0, 0]
        aw_ref[...] = aw
        part = lax.dot_general(aw, pg_ref[...], (((1,), (0,)), ((), ())),
                               preferred_element_type=jnp.float32)

        @pl.when(i == 0)
        def _():
            pv_ref[...] = jnp.zeros_like(pv_ref)

        pv_ref[...] += part

    return pl.pallas_call(
        body,
        grid=(steps,),
        in_specs=[
            pl.BlockSpec((1, BC), lambda i: (0, i)),
            pl.BlockSpec((1, 128), lambda i: (0, 0)),
            pl.BlockSpec((1, 128), lambda i: (0, 0)),
            pl.BlockSpec((BC, EXT), lambda i: (i, 0)),
        ],
        out_specs=[
            pl.BlockSpec((1, BC), lambda i: (0, i)),
            pl.BlockSpec((1, EXT), lambda i: (0, 0)),
        ],
        out_shape=[
            jax.ShapeDtypeStruct((1, E), jnp.float32),
            jax.ShapeDtypeStruct((1, EXT), jnp.float32),
        ],
    )(scores, mrow, zirow, pg_mat)


def _sc_partials(row1d, col1d, sc1d, m16, zi16, E, N):
    epw = E // _NW
    pad = ((epw + 15) // 16) * 16
    chunks = pad // 16
    mesh = plsc.VectorSubcoreMesh(core_axis_name="c", subcore_axis_name="s",
                                  num_cores=_NC, num_subcores=_NS)

    @functools.partial(
        pl.kernel,
        out_type=(jax.ShapeDtypeStruct((_NW * N,), jnp.float32),
                  jax.ShapeDtypeStruct((_NW * N,), jnp.float32)),
        mesh=mesh,
        compiler_params=pltpu.CompilerParams(needs_layout_passes=False),
        scratch_types=[
            pltpu.VMEM((pad,), jnp.int32),
            pltpu.VMEM((pad,), jnp.int32),
            pltpu.VMEM((pad,), jnp.float32),
            pltpu.VMEM((16,), jnp.int32),
            pltpu.VMEM((16,), jnp.float32),
            pltpu.VMEM((N,), jnp.float32),
            pltpu.VMEM((N,), jnp.float32),
            pltpu.VMEM((32,), jnp.float32),
        ],
    )
    def sck(row_hbm, col_hbm, sc_hbm, m_hbm, zi_hbm, deg_out, w_out,
            row_v, col_v, val_v, kbuf, cbuf, degh, wh, stat_v):
        wid = lax.axis_index("s") * _NC + lax.axis_index("c")
        base = wid * epw
        iota = lax.iota(jnp.int32, 16)
        zf = jnp.zeros((16,), jnp.float32)
        zi_ = jnp.zeros((16,), jnp.int32)

        row_v[pl.ds(pad - 16, 16)] = zi_
        col_v[pl.ds(pad - 16, 16)] = zi_
        val_v[pl.ds(pad - 16, 16)] = jnp.full((16,), -1e30, jnp.float32)
        pltpu.sync_copy(row_hbm.at[pl.ds(base, epw)], row_v.at[pl.ds(0, epw)])
        pltpu.sync_copy(col_hbm.at[pl.ds(base, epw)], col_v.at[pl.ds(0, epw)])
        pltpu.sync_copy(sc_hbm.at[pl.ds(base, epw)], val_v.at[pl.ds(0, epw)])
        pltpu.sync_copy(m_hbm, stat_v.at[pl.ds(0, 16)])
        pltpu.sync_copy(zi_hbm, stat_v.at[pl.ds(16, 16)])

        def zero_body(j, _):
            degh[pl.ds(j * 16, 16)] = zf
            wh[pl.ds(j * 16, 16)] = zf
            return 0

        lax.fori_loop(0, N // 16, zero_body, 0)
        mv = stat_v[pl.ds(0, 16)]
        ziv = stat_v[pl.ds(16, 16)]

        def accum(hist, keys, vals):
            sk, sv = plsc.sort_key_val(keys, vals)
            cs = plsc.cumsum(sv)
            kbuf[...] = sk
            cbuf[...] = cs
            knext = plsc.load_gather(kbuf, [jnp.minimum(iota + 1, 15)])
            kprev = plsc.load_gather(kbuf, [jnp.maximum(iota - 1, 0)])
            cprev = plsc.load_gather(cbuf, [jnp.maximum(iota - 1, 0)])
            mend = (sk != knext) | (iota == 15)
            mst = (sk != kprev) & (iota > 0)
            plsc.addupdate_scatter(hist, [sk], cs, mask=mend)
            plsc.addupdate_scatter(hist, [sk], -cprev, mask=mst)

        def edge_body(j, _):
            ks = col_v[pl.ds(j * 16, 16)]
            rw = row_v[pl.ds(j * 16, 16)]
            s = val_v[pl.ds(j * 16, 16)]
            e = jnp.exp(s - mv) * ziv
            accum(degh, ks, e)
            vw = jnp.where(ks == 0, e, 0.0)

            @pl.when(jnp.max(vw) > 0.0)
            def _():
                accum(wh, rw, vw)

            return 0

        lax.fori_loop(0, chunks, edge_body, 0)
        pltpu.sync_copy(degh, deg_out.at[pl.ds(wid * N, N)])
        pltpu.sync_copy(wh, w_out.at[pl.ds(wid * N, N)])

    return sck(row1d, col1d, sc1d, m16, zi16)


def _stage_e(deg_part, w_part, enc, W_gcn, b_gcn2, h2, phx, pv, W_comb,
             b_comb2, W_out, b_out2, W_wh, N, V, EXT):
    def body(degp_ref, wp_ref, enc_ref, wgcn_ref, bgcn_ref, h2_ref, phx_ref,
             pv_ref, wcomb_ref, bcomb_ref, wout_ref, bout_ref, wwh_ref,
             main_ref, ext_ref):
        deg = jnp.sum(degp_ref[...], axis=0, keepdims=True) + 1.0
        dinv = lax.rsqrt(deg)
        w = jnp.sum(wp_ref[...], axis=0, keepdims=True)
        u = dinv * w
        col = lax.broadcasted_iota(jnp.int32, (1, deg.shape[1]), 1)
        u = u + jnp.where(col == 0, dinv, 0.0)
        v = lax.dot_general(u, enc_ref[...], (((1,), (0,)), ((), ())),
                            preferred_element_type=jnp.float32)
        aa = lax.dot_general(v, wgcn_ref[...], (((1,), (1,)), ((), ())),
                             preferred_element_type=jnp.float32)
        dinv0 = dinv[:, 0:1]
        attn_applied = dinv0 * aa + bgcn_ref[...]
        p = jnp.sum(attn_applied * wwh_ref[...]) + phx_ref[0, 0]
        p_gen = jax.nn.sigmoid(p) + 1e-07
        atten_p = pv_ref[...] * (1.0 - p_gen + 1e-07)
        cat = jnp.concatenate([h2_ref[...], attn_applied], axis=1)
        comb = jnp.tanh(
            lax.dot_general(cat, wcomb_ref[...], (((1,), (1,)), ((), ())),
                            preferred_element_type=jnp.float32)
            + bcomb_ref[...])
        logits = lax.dot_general(comb, wout_ref[...], (((1,), (1,)), ((), ())),
                                 preferred_element_type=jnp.float32) \
            + bout_ref[...]
        mx = jnp.max(logits, axis=1, keepdims=True)
        ex = jnp.exp(logits - mx)
        ssum = jnp.sum(ex, axis=1, keepdims=True)
        main_ref[...] = jnp.log(ex / ssum * p_gen)
        ext_ref[...] = jnp.log(atten_p)

    return pl.pallas_call(
        body,
        out_shape=[
            jax.ShapeDtypeStruct((1, V), jnp.float32),
            jax.ShapeDtypeStruct((1, EXT), jnp.float32),
        ],
    )(deg_part, w_part, enc, W_gcn, b_gcn2, h2, phx, pv, W_comb, b_comb2,
      W_out, b_out2, W_wh)


def kernel(input, hidden_h, hidden_c, encoder_outputs, syn_embeddeds,
           edge_index, pg_mat, emb_table, W_attn, W_gcn, b_gcn, W_comb,
           b_comb, W_ih, W_hh, b_ih, b_hh, W_out, b_out, W_wh, W_ws, W_wx,
           b_wx):
    N, _ = encoder_outputs.shape
    E, _ = syn_embeddeds.shape
    EXT = pg_mat.shape[1]
    V = W_out.shape[0]

    emb3 = emb_table.reshape(emb_table.shape[0], 1, H)
    h0 = hidden_h.reshape(1, H)
    c0 = hidden_c.reshape(1, H)
    h2, c2, q, phx = _stage_a(
        input.astype(jnp.int32), emb3, h0, c0, W_ih, W_hh,
        b_ih.reshape(1, HID4), b_hh.reshape(1, HID4), W_attn,
        W_ws, W_wx, b_wx.reshape(1, 1))

    scores, mrow, zirow = _stage_b(q, syn_embeddeds, E, 3200)
    aw, pv = _stage_c(scores, mrow, zirow, pg_mat, E, EXT, 3200)

    row1d = edge_index[0].astype(jnp.int32)
    col1d = edge_index[1].astype(jnp.int32)
    m16 = mrow.reshape(128)[:16]
    zi16 = zirow.reshape(128)[:16]
    deg_part, w_part = _sc_partials(row1d, col1d, scores.reshape(E),
                                    m16, zi16, E, N)

    out_main, out_ext = _stage_e(
        deg_part.reshape(_NW, N), w_part.reshape(_NW, N), encoder_outputs,
        W_gcn, b_gcn.reshape(1, H), h2, phx, pv, W_comb,
        b_comb.reshape(1, H), W_out, b_out.reshape(1, V), W_wh, N, V, EXT)

    out = jnp.concatenate([out_main, out_ext], axis=1)
    return (out, h2.reshape(1, 1, H), c2.reshape(1, 1, H), aw)


# speedup vs baseline: 20.5879x; 1.1984x over previous
import functools

import jax
import jax.numpy as jnp
from jax import lax
from jax.experimental import pallas as pl
from jax.experimental.pallas import tpu as pltpu
from jax.experimental.pallas import tpu_sc as plsc

H = 256
HID4 = 4 * H

_NC = 2
_NS = 16
_NW = _NC * _NS


def _stage_a(emb_row, h0, c0, W_ih, W_hh, b_ih2, b_hh2, W_attn,
             W_ws, W_wx, b_wx2):
    def body(emb_ref, h_ref, c_ref, wih_ref, whh_ref, bih_ref,
             bhh_ref, wattn_ref, wws_ref, wwx_ref, bwx_ref,
             h2_ref, c2_ref, q_ref, phx_ref, h23_ref, c23_ref):
        x = emb_ref[...]
        h = h_ref[...]
        c = c_ref[...]
        g = (lax.dot_general(x, wih_ref[...], (((1,), (1,)), ((), ())),
                             preferred_element_type=jnp.float32)
             + lax.dot_general(h, whh_ref[...], (((1,), (1,)), ((), ())),
                               preferred_element_type=jnp.float32)
             + bih_ref[...] + bhh_ref[...])
        i_g = jax.nn.sigmoid(g[:, 0:H])
        f_g = jax.nn.sigmoid(g[:, H:2 * H])
        g_g = jnp.tanh(g[:, 2 * H:3 * H])
        o_g = jax.nn.sigmoid(g[:, 3 * H:4 * H])
        c2 = f_g * c + i_g * g_g
        h2 = o_g * jnp.tanh(c2)
        h2_ref[...] = h2
        c2_ref[...] = c2
        q_ref[...] = lax.dot_general(h2, wattn_ref[...],
                                     (((1,), (1,)), ((), ())),
                                     preferred_element_type=jnp.float32)
        phx = (jnp.sum(h2 * wws_ref[...]) + jnp.sum(x * wwx_ref[...])
               + bwx_ref[0, 0])
        phx_ref[...] = jnp.full((1, 1), 0.0, jnp.float32) + phx
        h23_ref[...] = h2.reshape(1, 1, H)
        c23_ref[...] = c2.reshape(1, 1, H)

    return pl.pallas_call(
        body,
        out_shape=[
            jax.ShapeDtypeStruct((1, H), jnp.float32),
            jax.ShapeDtypeStruct((1, H), jnp.float32),
            jax.ShapeDtypeStruct((1, H), jnp.float32),
            jax.ShapeDtypeStruct((1, 1), jnp.float32),
            jax.ShapeDtypeStruct((1, 1, H), jnp.float32),
            jax.ShapeDtypeStruct((1, 1, H), jnp.float32),
        ],
    )(emb_row, h0, c0, W_ih, W_hh, b_ih2, b_hh2, W_attn, W_ws,
      W_wx, b_wx2)


def _stage_b(q, syn, E, BE):
    steps = E // BE

    def body(q_ref, syn_ref, s_ref, m_ref, zi_ref, scr):
        i = pl.program_id(0)

        @pl.when(i == 0)
        def _():
            scr[0] = -jnp.inf
            scr[1] = 0.0

        s = lax.dot_general(q_ref[...], syn_ref[...],
                            (((1,), (1,)), ((), ())),
                            preferred_element_type=jnp.float32)
        s_ref[...] = s
        m_old = scr[0]
        z_old = scr[1]
        m_blk = jnp.max(s)
        m_new = jnp.maximum(m_old, m_blk)
        z_new = z_old * jnp.exp(m_old - m_new) + jnp.sum(jnp.exp(s - m_new))
        scr[0] = m_new
        scr[1] = z_new

        @pl.when(i == steps - 1)
        def _():
            m_ref[...] = jnp.full((1, 128), m_new, jnp.float32)
            zi_ref[...] = jnp.full((1, 128), 1.0 / z_new, jnp.float32)

    return pl.pallas_call(
        body,
        grid=(steps,),
        in_specs=[
            pl.BlockSpec((1, H), lambda i: (0, 0)),
            pl.BlockSpec((BE, H), lambda i: (i, 0)),
        ],
        out_specs=[
            pl.BlockSpec((1, BE), lambda i: (0, i)),
            pl.BlockSpec((1, 128), lambda i: (0, 0)),
            pl.BlockSpec((1, 128), lambda i: (0, 0)),
        ],
        out_shape=[
            jax.ShapeDtypeStruct((1, E), jnp.float32),
            jax.ShapeDtypeStruct((1, 128), jnp.float32),
            jax.ShapeDtypeStruct((1, 128), jnp.float32),
        ],
        scratch_shapes=[pltpu.SMEM((2,), jnp.float32)],
    )(q, syn)


def _stage_c(scores, mrow, zirow, pg_mat, E, EXT, BC):
    steps = E // BC

    def body(s_ref, m_ref, zi_ref, pg_ref, aw_ref, pv_ref):
        i = pl.program_id(0)
        aw = jnp.exp(s_ref[...] - m_ref[0, 0]) * zi_ref[0, 0]
        aw_ref[...] = aw
        part = lax.dot_general(aw, pg_ref[...], (((1,), (0,)), ((), ())),
                               preferred_element_type=jnp.float32)

        @pl.when(i == 0)
        def _():
            pv_ref[...] = jnp.zeros_like(pv_ref)

        pv_ref[...] += part

    return pl.pallas_call(
        body,
        grid=(steps,),
        in_specs=[
            pl.BlockSpec((1, BC), lambda i: (0, i)),
            pl.BlockSpec((1, 128), lambda i: (0, 0)),
            pl.BlockSpec((1, 128), lambda i: (0, 0)),
            pl.BlockSpec((BC, EXT), lambda i: (i, 0)),
        ],
        out_specs=[
            pl.BlockSpec((1, BC), lambda i: (0, i)),
            pl.BlockSpec((1, EXT), lambda i: (0, 0)),
        ],
        out_shape=[
            jax.ShapeDtypeStruct((1, E), jnp.float32),
            jax.ShapeDtypeStruct((1, EXT), jnp.float32),
        ],
    )(scores, mrow, zirow, pg_mat)


def _sc_partials(row1d, col1d, sc1d, mrow, zirow, E, N):
    epw = E // _NW
    pad = ((epw + 15) // 16) * 16
    chunks = pad // 16
    mesh = plsc.VectorSubcoreMesh(core_axis_name="c", subcore_axis_name="s",
                                  num_cores=_NC, num_subcores=_NS)

    @functools.partial(
        pl.kernel,
        out_type=(jax.ShapeDtypeStruct((_NW * N,), jnp.float32),
                  jax.ShapeDtypeStruct((_NW * N,), jnp.float32)),
        mesh=mesh,
        compiler_params=pltpu.CompilerParams(needs_layout_passes=False),
        scratch_types=[
            pltpu.VMEM((pad,), jnp.int32),
            pltpu.VMEM((pad,), jnp.int32),
            pltpu.VMEM((pad,), jnp.float32),
            pltpu.VMEM((16,), jnp.int32),
            pltpu.VMEM((16,), jnp.float32),
            pltpu.VMEM((N,), jnp.float32),
            pltpu.VMEM((N,), jnp.float32),
            pltpu.VMEM((256,), jnp.float32),
        ],
    )
    def sck(row_hbm, col_hbm, sc_hbm, m_hbm, zi_hbm, deg_out, w_out,
            row_v, col_v, val_v, kbuf, cbuf, degh, wh, stat_v):
        wid = lax.axis_index("s") * _NC + lax.axis_index("c")
        base = wid * epw
        iota = lax.iota(jnp.int32, 16)
        zf = jnp.zeros((16,), jnp.float32)
        zi_ = jnp.zeros((16,), jnp.int32)

        row_v[pl.ds(pad - 16, 16)] = zi_
        col_v[pl.ds(pad - 16, 16)] = zi_
        val_v[pl.ds(pad - 16, 16)] = jnp.full((16,), -1e30, jnp.float32)
        pltpu.sync_copy(row_hbm.at[pl.ds(base, epw)], row_v.at[pl.ds(0, epw)])
        pltpu.sync_copy(col_hbm.at[pl.ds(base, epw)], col_v.at[pl.ds(0, epw)])
        pltpu.sync_copy(sc_hbm.at[pl.ds(base, epw)], val_v.at[pl.ds(0, epw)])
        pltpu.sync_copy(m_hbm.at[0, pl.ds(0, 128)], stat_v.at[pl.ds(0, 128)])
        pltpu.sync_copy(zi_hbm.at[0, pl.ds(0, 128)],
                        stat_v.at[pl.ds(128, 128)])

        def zero_body(j, _):
            degh[pl.ds(j * 16, 16)] = zf
            wh[pl.ds(j * 16, 16)] = zf
            return 0

        lax.fori_loop(0, N // 16, zero_body, 0)
        mv = stat_v[pl.ds(0, 16)]
        ziv = stat_v[pl.ds(128, 16)]

        def accum(hist, keys, vals):
            sk, sv = plsc.sort_key_val(keys, vals)
            cs = plsc.cumsum(sv)
            kbuf[...] = sk
            cbuf[...] = cs
            knext = plsc.load_gather(kbuf, [jnp.minimum(iota + 1, 15)])
            kprev = plsc.load_gather(kbuf, [jnp.maximum(iota - 1, 0)])
            cprev = plsc.load_gather(cbuf, [jnp.maximum(iota - 1, 0)])
            mend = (sk != knext) | (iota == 15)
            mst = (sk != kprev) & (iota > 0)
            plsc.addupdate_scatter(hist, [sk], cs, mask=mend)
            plsc.addupdate_scatter(hist, [sk], -cprev, mask=mst)

        def edge_body(j, _):
            ks = col_v[pl.ds(j * 16, 16)]
            rw = row_v[pl.ds(j * 16, 16)]
            s = val_v[pl.ds(j * 16, 16)]
            e = jnp.exp(s - mv) * ziv
            accum(degh, ks, e)
            vw = jnp.where(ks == 0, e, 0.0)

            @pl.when(jnp.max(vw) > 0.0)
            def _():
                accum(wh, rw, vw)

            return 0

        lax.fori_loop(0, chunks, edge_body, 0)
        pltpu.sync_copy(degh, deg_out.at[pl.ds(wid * N, N)])
        pltpu.sync_copy(wh, w_out.at[pl.ds(wid * N, N)])

    return sck(row1d, col1d, sc1d, mrow, zirow)


def _stage_e(deg_part, w_part, enc, W_gcn, b_gcn2, h2, phx, pv, W_comb,
             b_comb2, W_out, b_out2, W_wh, N, V, EXT):
    def body(degp_ref, wp_ref, enc_ref, wgcn_ref, bgcn_ref, h2_ref, phx_ref,
             pv_ref, wcomb_ref, bcomb_ref, wout_ref, bout_ref, wwh_ref,
             main_ref, ext_ref):
        deg = jnp.sum(degp_ref[...], axis=0, keepdims=True) + 1.0
        dinv = lax.rsqrt(deg)
        w = jnp.sum(wp_ref[...], axis=0, keepdims=True)
        u = dinv * w
        col = lax.broadcasted_iota(jnp.int32, (1, deg.shape[1]), 1)
        u = u + jnp.where(col == 0, dinv, 0.0)
        v = lax.dot_general(u, enc_ref[...], (((1,), (0,)), ((), ())),
                            preferred_element_type=jnp.float32)
        aa = lax.dot_general(v, wgcn_ref[...], (((1,), (1,)), ((), ())),
                             preferred_element_type=jnp.float32)
        dinv0 = dinv[:, 0:1]
        attn_applied = dinv0 * aa + bgcn_ref[...]
        p = jnp.sum(attn_applied * wwh_ref[...]) + phx_ref[0, 0]
        p_gen = jax.nn.sigmoid(p) + 1e-07
        atten_p = pv_ref[...] * (1.0 - p_gen + 1e-07)
        cat = jnp.concatenate([h2_ref[...], attn_applied], axis=1)
        comb = jnp.tanh(
            lax.dot_general(cat, wcomb_ref[...], (((1,), (1,)), ((), ())),
                            preferred_element_type=jnp.float32)
            + bcomb_ref[...])
        logits = lax.dot_general(comb, wout_ref[...], (((1,), (1,)), ((), ())),
                                 preferred_element_type=jnp.float32) \
            + bout_ref[...]
        mx = jnp.max(logits, axis=1, keepdims=True)
        ex = jnp.exp(logits - mx)
        ssum = jnp.sum(ex, axis=1, keepdims=True)
        main_ref[...] = jnp.log(ex / ssum * p_gen)
        ext_ref[...] = jnp.log(atten_p)

    return pl.pallas_call(
        body,
        out_shape=[
            jax.ShapeDtypeStruct((1, V), jnp.float32),
            jax.ShapeDtypeStruct((1, EXT), jnp.float32),
        ],
    )(deg_part, w_part, enc, W_gcn, b_gcn2, h2, phx, pv, W_comb, b_comb2,
      W_out, b_out2, W_wh)


def kernel(input, hidden_h, hidden_c, encoder_outputs, syn_embeddeds,
           edge_index, pg_mat, emb_table, W_attn, W_gcn, b_gcn, W_comb,
           b_comb, W_ih, W_hh, b_ih, b_hh, W_out, b_out, W_wh, W_ws, W_wx,
           b_wx):
    N, _ = encoder_outputs.shape
    E, _ = syn_embeddeds.shape
    EXT = pg_mat.shape[1]
    V = W_out.shape[0]

    h0 = hidden_h.reshape(1, H)
    c0 = hidden_c.reshape(1, H)
    emb_row = jnp.take(emb_table, input.astype(jnp.int32), axis=0)
    h2, c2, q, phx, h23, c23 = _stage_a(
        emb_row, h0, c0, W_ih, W_hh,
        b_ih.reshape(1, HID4), b_hh.reshape(1, HID4), W_attn,
        W_ws, W_wx, b_wx.reshape(1, 1))

    scores, mrow, zirow = _stage_b(q, syn_embeddeds, E, 6400)
    aw, pv = _stage_c(scores, mrow, zirow, pg_mat, E, EXT, 3200)

    row1d = edge_index[0].astype(jnp.int32)
    col1d = edge_index[1].astype(jnp.int32)
    deg_part, w_part = _sc_partials(row1d, col1d, scores.reshape(E),
                                    mrow, zirow, E, N)

    out_main, out_ext = _stage_e(
        deg_part.reshape(_NW, N), w_part.reshape(_NW, N), encoder_outputs,
        W_gcn, b_gcn.reshape(1, H), h2, phx, pv, W_comb,
        b_comb.reshape(1, H), W_out, b_out.reshape(1, V), W_wh, N, V, EXT)

    out = jnp.concatenate([out_main, out_ext], axis=1)
    return (out, h23, c23, aw)


# speedup vs baseline: 21.1145x; 1.0256x over previous
import functools

import jax
import jax.numpy as jnp
from jax import lax
from jax.experimental import pallas as pl
from jax.experimental.pallas import tpu as pltpu
from jax.experimental.pallas import tpu_sc as plsc

H = 256
HID4 = 4 * H

_NC = 2
_NS = 16
_NW = _NC * _NS


def _stage_a(emb_row, h0, c0, W_ih, W_hh, b_ih2, b_hh2, W_attn,
             W_ws, W_wx, b_wx2):
    def body(emb_ref, h_ref, c_ref, wih_ref, whh_ref, bih_ref,
             bhh_ref, wattn_ref, wws_ref, wwx_ref, bwx_ref,
             h2_ref, c2_ref, q_ref, phx_ref, h23_ref, c23_ref):
        x = emb_ref[...]
        h = h_ref[...]
        c = c_ref[...]
        g = (lax.dot_general(x, wih_ref[...], (((1,), (1,)), ((), ())),
                             preferred_element_type=jnp.float32)
             + lax.dot_general(h, whh_ref[...], (((1,), (1,)), ((), ())),
                               preferred_element_type=jnp.float32)
             + bih_ref[...] + bhh_ref[...])
        i_g = jax.nn.sigmoid(g[:, 0:H])
        f_g = jax.nn.sigmoid(g[:, H:2 * H])
        g_g = jnp.tanh(g[:, 2 * H:3 * H])
        o_g = jax.nn.sigmoid(g[:, 3 * H:4 * H])
        c2 = f_g * c + i_g * g_g
        h2 = o_g * jnp.tanh(c2)
        h2_ref[...] = h2
        c2_ref[...] = c2
        q_ref[...] = lax.dot_general(h2, wattn_ref[...],
                                     (((1,), (1,)), ((), ())),
                                     preferred_element_type=jnp.float32)
        phx = (jnp.sum(h2 * wws_ref[...]) + jnp.sum(x * wwx_ref[...])
               + bwx_ref[0, 0])
        phx_ref[...] = jnp.full((1, 1), 0.0, jnp.float32) + phx
        h23_ref[...] = h2.reshape(1, 1, H)
        c23_ref[...] = c2.reshape(1, 1, H)

    return pl.pallas_call(
        body,
        out_shape=[
            jax.ShapeDtypeStruct((1, H), jnp.float32),
            jax.ShapeDtypeStruct((1, H), jnp.float32),
            jax.ShapeDtypeStruct((1, H), jnp.float32),
            jax.ShapeDtypeStruct((1, 1), jnp.float32),
            jax.ShapeDtypeStruct((1, 1, H), jnp.float32),
            jax.ShapeDtypeStruct((1, 1, H), jnp.float32),
        ],
    )(emb_row, h0, c0, W_ih, W_hh, b_ih2, b_hh2, W_attn, W_ws,
      W_wx, b_wx2)


def _stage_b(q, syn, E, BE):
    steps = E // BE

    def body(q_ref, syn_ref, s_ref, m_ref, zi_ref, scr):
        i = pl.program_id(0)

        @pl.when(i == 0)
        def _():
            scr[0] = -jnp.inf
            scr[1] = 0.0

        s = lax.dot_general(q_ref[...], syn_ref[...],
                            (((1,), (1,)), ((), ())),
                            preferred_element_type=jnp.float32)
        s_ref[...] = s
        m_old = scr[0]
        z_old = scr[1]
        m_blk = jnp.max(s)
        m_new = jnp.maximum(m_old, m_blk)
        z_new = z_old * jnp.exp(m_old - m_new) + jnp.sum(jnp.exp(s - m_new))
        scr[0] = m_new
        scr[1] = z_new

        @pl.when(i == steps - 1)
        def _():
            m_ref[...] = jnp.full((1, 128), m_new, jnp.float32)
            zi_ref[...] = jnp.full((1, 128), 1.0 / z_new, jnp.float32)

    return pl.pallas_call(
        body,
        grid=(steps,),
        in_specs=[
            pl.BlockSpec((1, H), lambda i: (0, 0)),
            pl.BlockSpec((BE, H), lambda i: (i, 0)),
        ],
        out_specs=[
            pl.BlockSpec((1, BE), lambda i: (0, i)),
            pl.BlockSpec((1, 128), lambda i: (0, 0)),
            pl.BlockSpec((1, 128), lambda i: (0, 0)),
        ],
        out_shape=[
            jax.ShapeDtypeStruct((1, E), jnp.float32),
            jax.ShapeDtypeStruct((1, 128), jnp.float32),
            jax.ShapeDtypeStruct((1, 128), jnp.float32),
        ],
        scratch_shapes=[pltpu.SMEM((2,), jnp.float32)],
    )(q, syn)


def _stage_c(scores, mrow, zirow, pg_mat, E, EXT, BC):
    steps = E // BC

    def body(s_ref, m_ref, zi_ref, pg_ref, aw_ref, pv_ref):
        i = pl.program_id(0)
        aw = jnp.exp(s_ref[...] - m_ref[0, 0]) * zi_ref[0, 0]
        aw_ref[...] = aw
        part = lax.dot_general(aw, pg_ref[...], (((1,), (0,)), ((), ())),
                               preferred_element_type=jnp.float32)

        @pl.when(i == 0)
        def _():
            pv_ref[...] = jnp.zeros_like(pv_ref)

        pv_ref[...] += part

    return pl.pallas_call(
        body,
        grid=(steps,),
        in_specs=[
            pl.BlockSpec((1, BC), lambda i: (0, i)),
            pl.BlockSpec((1, 128), lambda i: (0, 0)),
            pl.BlockSpec((1, 128), lambda i: (0, 0)),
            pl.BlockSpec((BC, EXT), lambda i: (i, 0)),
        ],
        out_specs=[
            pl.BlockSpec((1, BC), lambda i: (0, i)),
            pl.BlockSpec((1, EXT), lambda i: (0, 0)),
        ],
        out_shape=[
            jax.ShapeDtypeStruct((1, E), jnp.float32),
            jax.ShapeDtypeStruct((1, EXT), jnp.float32),
        ],
    )(scores, mrow, zirow, pg_mat)


def _sc_partials(row1d, col1d, sc1d, mrow, zirow, E, N):
    epw = E // _NW
    pad = ((epw + 15) // 16) * 16
    chunks = pad // 16
    NP = ((N + 127) // 128) * 128
    mesh = plsc.VectorSubcoreMesh(core_axis_name="c", subcore_axis_name="s",
                                  num_cores=_NC, num_subcores=_NS)

    @functools.partial(
        pl.kernel,
        out_type=(jax.ShapeDtypeStruct((_NW * NP,), jnp.float32),
                  jax.ShapeDtypeStruct((_NW * NP,), jnp.float32)),
        mesh=mesh,
        compiler_params=pltpu.CompilerParams(needs_layout_passes=False),
        scratch_types=[
            pltpu.VMEM((pad,), jnp.int32),
            pltpu.VMEM((pad,), jnp.int32),
            pltpu.VMEM((pad,), jnp.float32),
            pltpu.VMEM((16,), jnp.int32),
            pltpu.VMEM((16,), jnp.float32),
            pltpu.VMEM((NP,), jnp.float32),
            pltpu.VMEM((NP,), jnp.float32),
            pltpu.VMEM((256,), jnp.float32),
        ],
    )
    def sck(row_hbm, col_hbm, sc_hbm, m_hbm, zi_hbm, deg_out, w_out,
            row_v, col_v, val_v, kbuf, cbuf, degh, wh, stat_v):
        wid = lax.axis_index("s") * _NC + lax.axis_index("c")
        base = wid * epw
        iota = lax.iota(jnp.int32, 16)
        zf = jnp.zeros((16,), jnp.float32)
        zi_ = jnp.zeros((16,), jnp.int32)

        row_v[pl.ds(pad - 16, 16)] = zi_
        col_v[pl.ds(pad - 16, 16)] = zi_
        val_v[pl.ds(pad - 16, 16)] = jnp.full((16,), -1e30, jnp.float32)
        pltpu.sync_copy(row_hbm.at[pl.ds(base, epw)], row_v.at[pl.ds(0, epw)])
        pltpu.sync_copy(col_hbm.at[pl.ds(base, epw)], col_v.at[pl.ds(0, epw)])
        pltpu.sync_copy(sc_hbm.at[pl.ds(base, epw)], val_v.at[pl.ds(0, epw)])
        pltpu.sync_copy(m_hbm.at[0, pl.ds(0, 128)], stat_v.at[pl.ds(0, 128)])
        pltpu.sync_copy(zi_hbm.at[0, pl.ds(0, 128)],
                        stat_v.at[pl.ds(128, 128)])

        def zero_body(j, _):
            degh[pl.ds(j * 16, 16)] = zf
            wh[pl.ds(j * 16, 16)] = zf
            return 0

        lax.fori_loop(0, NP // 16, zero_body, 0)
        mv = stat_v[pl.ds(0, 16)]
        ziv = stat_v[pl.ds(128, 16)]

        def accum(hist, keys, vals):
            sk, sv = plsc.sort_key_val(keys, vals)
            cs = plsc.cumsum(sv)
            kbuf[...] = sk
            cbuf[...] = cs
            knext = plsc.load_gather(kbuf, [jnp.minimum(iota + 1, 15)])
            kprev = plsc.load_gather(kbuf, [jnp.maximum(iota - 1, 0)])
            cprev = plsc.load_gather(cbuf, [jnp.maximum(iota - 1, 0)])
            mend = (sk != knext) | (iota == 15)
            mst = (sk != kprev) & (iota > 0)
            plsc.addupdate_scatter(hist, [sk], cs, mask=mend)
            plsc.addupdate_scatter(hist, [sk], -cprev, mask=mst)

        def edge_body(j, _):
            ks = col_v[pl.ds(j * 16, 16)]
            rw = row_v[pl.ds(j * 16, 16)]
            s = val_v[pl.ds(j * 16, 16)]
            e = jnp.exp(s - mv) * ziv
            accum(degh, ks, e)
            vw = jnp.where(ks == 0, e, 0.0)

            @pl.when(jnp.max(vw) > 0.0)
            def _():
                accum(wh, rw, vw)

            return 0

        lax.fori_loop(0, chunks, edge_body, 0)
        pltpu.sync_copy(degh, deg_out.at[pl.ds(wid * NP, NP)])
        pltpu.sync_copy(wh, w_out.at[pl.ds(wid * NP, NP)])

    return sck(row1d, col1d, sc1d, mrow, zirow)


def _stage_e(deg_part, w_part, enc, W_gcn, b_gcn2, h2, phx, pv, W_comb,
             b_comb2, W_out, b_out2, W_wh, N, V, EXT):
    def body(degp_ref, wp_ref, enc_ref, wgcn_ref, bgcn_ref, h2_ref, phx_ref,
             pv_ref, wcomb_ref, bcomb_ref, wout_ref, bout_ref, wwh_ref,
             main_ref, ext_ref):
        deg = jnp.sum(degp_ref[...], axis=0, keepdims=True) + 1.0
        dinv = lax.rsqrt(deg)
        w = jnp.sum(wp_ref[...], axis=0, keepdims=True)
        u = dinv * w
        col = lax.broadcasted_iota(jnp.int32, (1, deg.shape[1]), 1)
        u = u + jnp.where(col == 0, dinv, 0.0)
        u = u[:, :enc_ref.shape[0]]
        v = lax.dot_general(u, enc_ref[...], (((1,), (0,)), ((), ())),
                            preferred_element_type=jnp.float32)
        aa = lax.dot_general(v, wgcn_ref[...], (((1,), (1,)), ((), ())),
                             preferred_element_type=jnp.float32)
        dinv0 = dinv[:, 0:1]
        attn_applied = dinv0 * aa + bgcn_ref[...]
        p = jnp.sum(attn_applied * wwh_ref[...]) + phx_ref[0, 0]
        p_gen = jax.nn.sigmoid(p) + 1e-07
        atten_p = pv_ref[...] * (1.0 - p_gen + 1e-07)
        cat = jnp.concatenate([h2_ref[...], attn_applied], axis=1)
        comb = jnp.tanh(
            lax.dot_general(cat, wcomb_ref[...], (((1,), (1,)), ((), ())),
                            preferred_element_type=jnp.float32)
            + bcomb_ref[...])
        logits = lax.dot_general(comb, wout_ref[...], (((1,), (1,)), ((), ())),
                                 preferred_element_type=jnp.float32) \
            + bout_ref[...]
        mx = jnp.max(logits, axis=1, keepdims=True)
        ex = jnp.exp(logits - mx)
        ssum = jnp.sum(ex, axis=1, keepdims=True)
        main_ref[...] = jnp.log(ex / ssum * p_gen)
        ext_ref[...] = jnp.log(atten_p)

    return pl.pallas_call(
        body,
        out_shape=[
            jax.ShapeDtypeStruct((1, V), jnp.float32),
            jax.ShapeDtypeStruct((1, EXT), jnp.float32),
        ],
    )(deg_part, w_part, enc, W_gcn, b_gcn2, h2, phx, pv, W_comb, b_comb2,
      W_out, b_out2, W_wh)


def kernel(input, hidden_h, hidden_c, encoder_outputs, syn_embeddeds,
           edge_index, pg_mat, emb_table, W_attn, W_gcn, b_gcn, W_comb,
           b_comb, W_ih, W_hh, b_ih, b_hh, W_out, b_out, W_wh, W_ws, W_wx,
           b_wx):
    N, _ = encoder_outputs.shape
    E, _ = syn_embeddeds.shape
    EXT = pg_mat.shape[1]
    V = W_out.shape[0]

    h0 = hidden_h.reshape(1, H)
    c0 = hidden_c.reshape(1, H)
    emb_row = jnp.take(emb_table, input.astype(jnp.int32), axis=0)
    h2, c2, q, phx, h23, c23 = _stage_a(
        emb_row, h0, c0, W_ih, W_hh,
        b_ih.reshape(1, HID4), b_hh.reshape(1, HID4), W_attn,
        W_ws, W_wx, b_wx.reshape(1, 1))

    scores, mrow, zirow = _stage_b(q, syn_embeddeds, E, 16000)
    aw, pv = _stage_c(scores, mrow, zirow, pg_mat, E, EXT, 3200)

    row1d = edge_index[0].astype(jnp.int32)
    col1d = edge_index[1].astype(jnp.int32)
    deg_part, w_part = _sc_partials(row1d, col1d, scores.reshape(E),
                                    mrow, zirow, E, N)
    NP = deg_part.shape[0] // _NW

    out_main, out_ext = _stage_e(
        deg_part.reshape(_NW, NP), w_part.reshape(_NW, NP), encoder_outputs,
        W_gcn, b_gcn.reshape(1, H), h2, phx, pv, W_comb,
        b_comb.reshape(1, H), W_out, b_out.reshape(1, V), W_wh, N, V, EXT)

    out = jnp.concatenate([out_main, out_ext], axis=1)
    return (out, h23, c23, aw)


# speedup vs baseline: 22.6981x; 1.0750x over previous
import functools

import jax
import jax.numpy as jnp
from jax import lax
from jax.experimental import pallas as pl
from jax.experimental.pallas import tpu as pltpu
from jax.experimental.pallas import tpu_sc as plsc

H = 256
HID4 = 4 * H

_NC = 2
_NS = 16
_NW = _NC * _NS


def _stage_a(emb_row, h0, c0, W_ih, W_hh, b_ih2, b_hh2, W_attn,
             W_ws, W_wx, b_wx2):
    def body(emb_ref, h_ref, c_ref, wih_ref, whh_ref, bih_ref,
             bhh_ref, wattn_ref, wws_ref, wwx_ref, bwx_ref,
             h2_ref, c2_ref, q_ref, phx_ref, h23_ref, c23_ref):
        x = emb_ref[...]
        h = h_ref[...]
        c = c_ref[...]
        g = (lax.dot_general(x, wih_ref[...], (((1,), (1,)), ((), ())),
                             preferred_element_type=jnp.float32)
             + lax.dot_general(h, whh_ref[...], (((1,), (1,)), ((), ())),
                               preferred_element_type=jnp.float32)
             + bih_ref[...] + bhh_ref[...])
        i_g = jax.nn.sigmoid(g[:, 0:H])
        f_g = jax.nn.sigmoid(g[:, H:2 * H])
        g_g = jnp.tanh(g[:, 2 * H:3 * H])
        o_g = jax.nn.sigmoid(g[:, 3 * H:4 * H])
        c2 = f_g * c + i_g * g_g
        h2 = o_g * jnp.tanh(c2)
        h2_ref[...] = h2
        c2_ref[...] = c2
        q_ref[...] = lax.dot_general(h2, wattn_ref[...],
                                     (((1,), (1,)), ((), ())),
                                     preferred_element_type=jnp.float32)
        phx = (jnp.sum(h2 * wws_ref[...]) + jnp.sum(x * wwx_ref[...])
               + bwx_ref[0, 0])
        phx_ref[...] = jnp.full((1, 1), 0.0, jnp.float32) + phx
        h23_ref[...] = h2.reshape(1, 1, H)
        c23_ref[...] = c2.reshape(1, 1, H)

    return pl.pallas_call(
        body,
        out_shape=[
            jax.ShapeDtypeStruct((1, H), jnp.float32),
            jax.ShapeDtypeStruct((1, H), jnp.float32),
            jax.ShapeDtypeStruct((1, H), jnp.float32),
            jax.ShapeDtypeStruct((1, 1), jnp.float32),
            jax.ShapeDtypeStruct((1, 1, H), jnp.float32),
            jax.ShapeDtypeStruct((1, 1, H), jnp.float32),
        ],
    )(emb_row, h0, c0, W_ih, W_hh, b_ih2, b_hh2, W_attn, W_ws,
      W_wx, b_wx2)


def _stage_b(q, syn, E, BE):
    steps = E // BE

    def body(q_ref, syn_ref, s_ref, m_ref, zi_ref, scr):
        i = pl.program_id(0)

        @pl.when(i == 0)
        def _():
            scr[0] = -jnp.inf
            scr[1] = 0.0

        s = lax.dot_general(q_ref[...], syn_ref[...],
                            (((1,), (1,)), ((), ())),
                            preferred_element_type=jnp.float32)
        s_ref[...] = s
        m_old = scr[0]
        z_old = scr[1]
        m_blk = jnp.max(s)
        m_new = jnp.maximum(m_old, m_blk)
        z_new = z_old * jnp.exp(m_old - m_new) + jnp.sum(jnp.exp(s - m_new))
        scr[0] = m_new
        scr[1] = z_new

        @pl.when(i == steps - 1)
        def _():
            m_ref[...] = jnp.full((1, 128), m_new, jnp.float32)
            zi_ref[...] = jnp.full((1, 128), 1.0 / z_new, jnp.float32)

    return pl.pallas_call(
        body,
        grid=(steps,),
        in_specs=[
            pl.BlockSpec((1, H), lambda i: (0, 0)),
            pl.BlockSpec((BE, H), lambda i: (i, 0)),
        ],
        out_specs=[
            pl.BlockSpec((1, BE), lambda i: (0, i)),
            pl.BlockSpec((1, 128), lambda i: (0, 0)),
            pl.BlockSpec((1, 128), lambda i: (0, 0)),
        ],
        out_shape=[
            jax.ShapeDtypeStruct((1, E), jnp.float32),
            jax.ShapeDtypeStruct((1, 128), jnp.float32),
            jax.ShapeDtypeStruct((1, 128), jnp.float32),
        ],
        scratch_shapes=[pltpu.SMEM((2,), jnp.float32)],
    )(q, syn)


def _stage_c(scores, mrow, zirow, pg_mat, E, EXT, BC):
    steps = E // BC

    def body(s_ref, m_ref, zi_ref, pg_ref, aw_ref, pv_ref):
        i = pl.program_id(0)
        aw = jnp.exp(s_ref[...] - m_ref[0, 0]) * zi_ref[0, 0]
        aw_ref[...] = aw
        part = lax.dot_general(aw, pg_ref[...], (((1,), (0,)), ((), ())),
                               preferred_element_type=jnp.float32)

        @pl.when(i == 0)
        def _():
            pv_ref[...] = jnp.zeros_like(pv_ref)

        pv_ref[...] += part

    return pl.pallas_call(
        body,
        grid=(steps,),
        in_specs=[
            pl.BlockSpec((1, BC), lambda i: (0, i)),
            pl.BlockSpec((1, 128), lambda i: (0, 0)),
            pl.BlockSpec((1, 128), lambda i: (0, 0)),
            pl.BlockSpec((BC, EXT), lambda i: (i, 0)),
        ],
        out_specs=[
            pl.BlockSpec((1, BC), lambda i: (0, i)),
            pl.BlockSpec((1, EXT), lambda i: (0, 0)),
        ],
        out_shape=[
            jax.ShapeDtypeStruct((1, E), jnp.float32),
            jax.ShapeDtypeStruct((1, EXT), jnp.float32),
        ],
    )(scores, mrow, zirow, pg_mat)


def _sc_partials(edge2d, scores2d, mrow, zirow, E, N):
    grains = E // 128
    base_len = (grains // _NW) * 128
    last_len = E - base_len * (_NW - 1)
    pad = last_len
    chunks = pad // 16
    NP = ((N + 127) // 128) * 128
    mesh = plsc.VectorSubcoreMesh(core_axis_name="c", subcore_axis_name="s",
                                  num_cores=_NC, num_subcores=_NS)

    @functools.partial(
        pl.kernel,
        out_type=(jax.ShapeDtypeStruct((_NW * NP,), jnp.float32),
                  jax.ShapeDtypeStruct((_NW * NP,), jnp.float32)),
        mesh=mesh,
        compiler_params=pltpu.CompilerParams(needs_layout_passes=False),
        scratch_types=[
            pltpu.VMEM((2, pad), jnp.int32),
            pltpu.VMEM((pad,), jnp.float32),
            pltpu.VMEM((16,), jnp.int32),
            pltpu.VMEM((16,), jnp.float32),
            pltpu.VMEM((NP,), jnp.float32),
            pltpu.VMEM((NP,), jnp.float32),
            pltpu.VMEM((256,), jnp.float32),
        ],
    )
    def sck(edge_hbm, sc_hbm, m_hbm, zi_hbm, deg_out, w_out,
            edge_v, val_v, kbuf, cbuf, degh, wh, stat_v):
        wid = lax.axis_index("s") * _NC + lax.axis_index("c")
        base = wid * base_len
        iota = lax.iota(jnp.int32, 16)
        zf = jnp.zeros((16,), jnp.float32)
        zi_ = jnp.zeros((16,), jnp.int32)
        neg = jnp.full((16,), -1e30, jnp.float32)

        def fill_body(j, _):
            off = base_len + j * 16
            edge_v[0, pl.ds(off, 16)] = zi_
            edge_v[1, pl.ds(off, 16)] = zi_
            val_v[pl.ds(off, 16)] = neg
            return 0

        lax.fori_loop(0, (pad - base_len) // 16, fill_body, 0)

        @pl.when(wid == _NW - 1)
        def _():
            pltpu.sync_copy(edge_hbm.at[pl.ds(0, 2), pl.ds(base, last_len)],
                            edge_v.at[pl.ds(0, 2), pl.ds(0, last_len)])
            pltpu.sync_copy(sc_hbm.at[0, pl.ds(base, last_len)],
                            val_v.at[pl.ds(0, last_len)])

        @pl.when(wid < _NW - 1)
        def _():
            pltpu.sync_copy(edge_hbm.at[pl.ds(0, 2), pl.ds(base, base_len)],
                            edge_v.at[pl.ds(0, 2), pl.ds(0, base_len)])
            pltpu.sync_copy(sc_hbm.at[0, pl.ds(base, base_len)],
                            val_v.at[pl.ds(0, base_len)])

        pltpu.sync_copy(m_hbm.at[0, pl.ds(0, 128)], stat_v.at[pl.ds(0, 128)])
        pltpu.sync_copy(zi_hbm.at[0, pl.ds(0, 128)],
                        stat_v.at[pl.ds(128, 128)])

        def zero_body(j, _):
            degh[pl.ds(j * 16, 16)] = zf
            wh[pl.ds(j * 16, 16)] = zf
            return 0

        lax.fori_loop(0, NP // 16, zero_body, 0)
        mv = stat_v[pl.ds(0, 16)]
        ziv = stat_v[pl.ds(128, 16)]

        def accum(hist, keys, vals):
            sk, sv = plsc.sort_key_val(keys, vals)
            cs = plsc.cumsum(sv)
            kbuf[...] = sk
            cbuf[...] = cs
            knext = plsc.load_gather(kbuf, [jnp.minimum(iota + 1, 15)])
            kprev = plsc.load_gather(kbuf, [jnp.maximum(iota - 1, 0)])
            cprev = plsc.load_gather(cbuf, [jnp.maximum(iota - 1, 0)])
            mend = (sk != knext) | (iota == 15)
            mst = (sk != kprev) & (iota > 0)
            plsc.addupdate_scatter(hist, [sk], cs, mask=mend)
            plsc.addupdate_scatter(hist, [sk], -cprev, mask=mst)

        def edge_body(j, _):
            ks = edge_v[1, pl.ds(j * 16, 16)]
            rw = edge_v[0, pl.ds(j * 16, 16)]
            s = val_v[pl.ds(j * 16, 16)]
            e = jnp.exp(s - mv) * ziv
            accum(degh, ks, e)
            vw = jnp.where(ks == 0, e, 0.0)

            @pl.when(jnp.max(vw) > 0.0)
            def _():
                accum(wh, rw, vw)

            return 0

        lax.fori_loop(0, chunks, edge_body, 0)
        pltpu.sync_copy(degh, deg_out.at[pl.ds(wid * NP, NP)])
        pltpu.sync_copy(wh, w_out.at[pl.ds(wid * NP, NP)])

    return sck(edge2d, scores2d, mrow, zirow)


def _stage_e(deg_part, w_part, enc, W_gcn, b_gcn2, h2, phx, pv, W_comb,
             b_comb2, W_out, b_out2, W_wh, N, V, EXT):
    def body(degp_ref, wp_ref, enc_ref, wgcn_ref, bgcn_ref, h2_ref, phx_ref,
             pv_ref, wcomb_ref, bcomb_ref, wout_ref, bout_ref, wwh_ref,
             main_ref, ext_ref):
        deg = jnp.sum(degp_ref[...], axis=0, keepdims=True) + 1.0
        dinv = lax.rsqrt(deg)
        w = jnp.sum(wp_ref[...], axis=0, keepdims=True)
        u = dinv * w
        col = lax.broadcasted_iota(jnp.int32, (1, deg.shape[1]), 1)
        u = u + jnp.where(col == 0, dinv, 0.0)
        u = u[:, :enc_ref.shape[0]]
        v = lax.dot_general(u, enc_ref[...], (((1,), (0,)), ((), ())),
                            preferred_element_type=jnp.float32)
        aa = lax.dot_general(v, wgcn_ref[...], (((1,), (1,)), ((), ())),
                             preferred_element_type=jnp.float32)
        dinv0 = dinv[:, 0:1]
        attn_applied = dinv0 * aa + bgcn_ref[...]
        p = jnp.sum(attn_applied * wwh_ref[...]) + phx_ref[0, 0]
        p_gen = jax.nn.sigmoid(p) + 1e-07
        atten_p = pv_ref[...] * (1.0 - p_gen + 1e-07)
        cat = jnp.concatenate([h2_ref[...], attn_applied], axis=1)
        comb = jnp.tanh(
            lax.dot_general(cat, wcomb_ref[...], (((1,), (1,)), ((), ())),
                            preferred_element_type=jnp.float32)
            + bcomb_ref[...])
        logits = lax.dot_general(comb, wout_ref[...], (((1,), (1,)), ((), ())),
                                 preferred_element_type=jnp.float32) \
            + bout_ref[...]
        mx = jnp.max(logits, axis=1, keepdims=True)
        ex = jnp.exp(logits - mx)
        ssum = jnp.sum(ex, axis=1, keepdims=True)
        main_ref[...] = jnp.log(ex / ssum * p_gen)
        ext_ref[...] = jnp.log(atten_p)

    return pl.pallas_call(
        body,
        out_shape=[
            jax.ShapeDtypeStruct((1, V), jnp.float32),
            jax.ShapeDtypeStruct((1, EXT), jnp.float32),
        ],
    )(deg_part, w_part, enc, W_gcn, b_gcn2, h2, phx, pv, W_comb, b_comb2,
      W_out, b_out2, W_wh)


def kernel(input, hidden_h, hidden_c, encoder_outputs, syn_embeddeds,
           edge_index, pg_mat, emb_table, W_attn, W_gcn, b_gcn, W_comb,
           b_comb, W_ih, W_hh, b_ih, b_hh, W_out, b_out, W_wh, W_ws, W_wx,
           b_wx):
    N, _ = encoder_outputs.shape
    E, _ = syn_embeddeds.shape
    EXT = pg_mat.shape[1]
    V = W_out.shape[0]

    h0 = hidden_h.reshape(1, H)
    c0 = hidden_c.reshape(1, H)
    emb_row = jnp.take(emb_table, input.astype(jnp.int32), axis=0)
    h2, c2, q, phx, h23, c23 = _stage_a(
        emb_row, h0, c0, W_ih, W_hh,
        b_ih.reshape(1, HID4), b_hh.reshape(1, HID4), W_attn,
        W_ws, W_wx, b_wx.reshape(1, 1))

    scores, mrow, zirow = _stage_b(q, syn_embeddeds, E, 16000)
    aw, pv = _stage_c(scores, mrow, zirow, pg_mat, E, EXT, 3200)

    deg_part, w_part = _sc_partials(edge_index.astype(jnp.int32), scores,
                                    mrow, zirow, E, N)
    NP = deg_part.shape[0] // _NW

    out_main, out_ext = _stage_e(
        deg_part.reshape(_NW, NP), w_part.reshape(_NW, NP), encoder_outputs,
        W_gcn, b_gcn.reshape(1, H), h2, phx, pv, W_comb,
        b_comb.reshape(1, H), W_out, b_out.reshape(1, V), W_wh, N, V, EXT)

    out = jnp.concatenate([out_main, out_ext], axis=1)
    return (out, h23, c23, aw)


# speedup vs baseline: 23.3476x; 1.0286x over previous
import functools

import jax
import jax.numpy as jnp
from jax import lax
from jax.experimental import pallas as pl
from jax.experimental.pallas import tpu as pltpu
from jax.experimental.pallas import tpu_sc as plsc

H = 256
HID4 = 4 * H

_NC = 2
_NS = 16
_NW = _NC * _NS


def _stage_a(emb_row, h0, c0, W_ih, W_hh, b_ih2, b_hh2, W_attn,
             W_ws, W_wx, b_wx2):
    def body(emb_ref, h_ref, c_ref, wih_ref, whh_ref, bih_ref,
             bhh_ref, wattn_ref, wws_ref, wwx_ref, bwx_ref,
             h2_ref, c2_ref, q_ref, phx_ref, h23_ref, c23_ref):
        x = emb_ref[...]
        h = h_ref[...]
        c = c_ref[...]
        g = (lax.dot_general(x, wih_ref[...], (((1,), (1,)), ((), ())),
                             preferred_element_type=jnp.float32)
             + lax.dot_general(h, whh_ref[...], (((1,), (1,)), ((), ())),
                               preferred_element_type=jnp.float32)
             + bih_ref[...] + bhh_ref[...])
        i_g = jax.nn.sigmoid(g[:, 0:H])
        f_g = jax.nn.sigmoid(g[:, H:2 * H])
        g_g = jnp.tanh(g[:, 2 * H:3 * H])
        o_g = jax.nn.sigmoid(g[:, 3 * H:4 * H])
        c2 = f_g * c + i_g * g_g
        h2 = o_g * jnp.tanh(c2)
        h2_ref[...] = h2
        c2_ref[...] = c2
        q_ref[...] = lax.dot_general(h2, wattn_ref[...],
                                     (((1,), (1,)), ((), ())),
                                     preferred_element_type=jnp.float32)
        phx = (jnp.sum(h2 * wws_ref[...]) + jnp.sum(x * wwx_ref[...])
               + bwx_ref[0, 0])
        phx_ref[...] = jnp.full((1, 1), 0.0, jnp.float32) + phx
        h23_ref[...] = h2.reshape(1, 1, H)
        c23_ref[...] = c2.reshape(1, 1, H)

    return pl.pallas_call(
        body,
        out_shape=[
            jax.ShapeDtypeStruct((1, H), jnp.float32),
            jax.ShapeDtypeStruct((1, H), jnp.float32),
            jax.ShapeDtypeStruct((1, H), jnp.float32),
            jax.ShapeDtypeStruct((1, 1), jnp.float32),
            jax.ShapeDtypeStruct((1, 1, H), jnp.float32),
            jax.ShapeDtypeStruct((1, 1, H), jnp.float32),
        ],
    )(emb_row, h0, c0, W_ih, W_hh, b_ih2, b_hh2, W_attn, W_ws,
      W_wx, b_wx2)


def _stage_b(q, syn, E, BE):
    steps = E // BE

    def body(q_ref, syn_ref, s_ref, m_ref, zi_ref, scr):
        i = pl.program_id(0)

        @pl.when(i == 0)
        def _():
            scr[0] = -jnp.inf
            scr[1] = 0.0

        s = lax.dot_general(q_ref[...], syn_ref[...],
                            (((1,), (1,)), ((), ())),
                            preferred_element_type=jnp.float32)
        s_ref[...] = s
        m_old = scr[0]
        z_old = scr[1]
        m_blk = jnp.max(s)
        m_new = jnp.maximum(m_old, m_blk)
        z_new = z_old * jnp.exp(m_old - m_new) + jnp.sum(jnp.exp(s - m_new))
        scr[0] = m_new
        scr[1] = z_new

        @pl.when(i == steps - 1)
        def _():
            m_ref[...] = jnp.full((1, 128), m_new, jnp.float32)
            zi_ref[...] = jnp.full((1, 128), 1.0 / z_new, jnp.float32)

    return pl.pallas_call(
        body,
        grid=(steps,),
        in_specs=[
            pl.BlockSpec((1, H), lambda i: (0, 0)),
            pl.BlockSpec((BE, H), lambda i: (i, 0)),
        ],
        out_specs=[
            pl.BlockSpec((1, BE), lambda i: (0, i)),
            pl.BlockSpec((1, 128), lambda i: (0, 0)),
            pl.BlockSpec((1, 128), lambda i: (0, 0)),
        ],
        out_shape=[
            jax.ShapeDtypeStruct((1, E), jnp.float32),
            jax.ShapeDtypeStruct((1, 128), jnp.float32),
            jax.ShapeDtypeStruct((1, 128), jnp.float32),
        ],
        scratch_shapes=[pltpu.SMEM((2,), jnp.float32)],
    )(q, syn)


def _stage_c(scores, mrow, zirow, pg_mat, E, EXT, BC):
    steps = E // BC

    def body(s_ref, m_ref, zi_ref, pg_ref, aw_ref, pv_ref):
        i = pl.program_id(0)
        aw = jnp.exp(s_ref[...] - m_ref[0, 0]) * zi_ref[0, 0]
        aw_ref[...] = aw
        part = lax.dot_general(aw, pg_ref[...], (((1,), (0,)), ((), ())),
                               preferred_element_type=jnp.float32)

        @pl.when(i == 0)
        def _():
            pv_ref[...] = jnp.zeros_like(pv_ref)

        pv_ref[...] += part

    return pl.pallas_call(
        body,
        grid=(steps,),
        in_specs=[
            pl.BlockSpec((1, BC), lambda i: (0, i)),
            pl.BlockSpec((1, 128), lambda i: (0, 0)),
            pl.BlockSpec((1, 128), lambda i: (0, 0)),
            pl.BlockSpec((BC, EXT), lambda i: (i, 0)),
        ],
        out_specs=[
            pl.BlockSpec((1, BC), lambda i: (0, i)),
            pl.BlockSpec((1, EXT), lambda i: (0, 0)),
        ],
        out_shape=[
            jax.ShapeDtypeStruct((1, E), jnp.float32),
            jax.ShapeDtypeStruct((1, EXT), jnp.float32),
        ],
    )(scores, mrow, zirow, pg_mat)


def _sc_partials(edge2d, scores2d, mrow, zirow, E, N):
    grains = E // 128
    base_len = (grains // _NW) * 128
    last_len = E - base_len * (_NW - 1)
    pad = last_len
    chunks = pad // 16
    NP = ((N + 127) // 128) * 128
    mesh = plsc.VectorSubcoreMesh(core_axis_name="c", subcore_axis_name="s",
                                  num_cores=_NC, num_subcores=_NS)

    @functools.partial(
        pl.kernel,
        out_type=(jax.ShapeDtypeStruct((_NW * NP,), jnp.float32),
                  jax.ShapeDtypeStruct((_NW * NP,), jnp.float32)),
        mesh=mesh,
        compiler_params=pltpu.CompilerParams(needs_layout_passes=False),
        scratch_types=[
            pltpu.VMEM((2, pad), jnp.int32),
            pltpu.VMEM((pad,), jnp.float32),
            pltpu.VMEM((16,), jnp.int32),
            pltpu.VMEM((16,), jnp.float32),
            pltpu.VMEM((NP,), jnp.float32),
            pltpu.VMEM((NP,), jnp.float32),
            pltpu.VMEM((256,), jnp.float32),
        ],
    )
    def sck(edge_hbm, sc_hbm, m_hbm, zi_hbm, deg_out, w_out,
            edge_v, val_v, kbuf, cbuf, degh, wh, stat_v):
        wid = lax.axis_index("s") * _NC + lax.axis_index("c")
        base = wid * base_len
        iota = lax.iota(jnp.int32, 16)
        zf = jnp.zeros((16,), jnp.float32)
        zi_ = jnp.zeros((16,), jnp.int32)
        neg = jnp.full((16,), -1e30, jnp.float32)

        def fill_body(j, _):
            off = base_len + j * 16
            edge_v[0, pl.ds(off, 16)] = zi_
            edge_v[1, pl.ds(off, 16)] = zi_
            val_v[pl.ds(off, 16)] = neg
            return 0

        lax.fori_loop(0, (pad - base_len) // 16, fill_body, 0)

        @pl.when(wid == _NW - 1)
        def _():
            pltpu.sync_copy(edge_hbm.at[pl.ds(0, 2), pl.ds(base, last_len)],
                            edge_v.at[pl.ds(0, 2), pl.ds(0, last_len)])
            pltpu.sync_copy(sc_hbm.at[0, pl.ds(base, last_len)],
                            val_v.at[pl.ds(0, last_len)])

        @pl.when(wid < _NW - 1)
        def _():
            pltpu.sync_copy(edge_hbm.at[pl.ds(0, 2), pl.ds(base, base_len)],
                            edge_v.at[pl.ds(0, 2), pl.ds(0, base_len)])
            pltpu.sync_copy(sc_hbm.at[0, pl.ds(base, base_len)],
                            val_v.at[pl.ds(0, base_len)])

        pltpu.sync_copy(m_hbm.at[0, pl.ds(0, 128)], stat_v.at[pl.ds(0, 128)])
        pltpu.sync_copy(zi_hbm.at[0, pl.ds(0, 128)],
                        stat_v.at[pl.ds(128, 128)])

        def zero_body(j, _):
            degh[pl.ds(j * 16, 16)] = zf
            wh[pl.ds(j * 16, 16)] = zf
            return 0

        lax.fori_loop(0, NP // 16, zero_body, 0)
        mv = stat_v[pl.ds(0, 16)]
        ziv = stat_v[pl.ds(128, 16)]

        def accum(hist, keys, vals):
            sk, sv = plsc.sort_key_val(keys, vals)
            cs = plsc.cumsum(sv)
            kbuf[...] = sk
            cbuf[...] = cs
            knext = plsc.load_gather(kbuf, [jnp.minimum(iota + 1, 15)])
            kprev = plsc.load_gather(kbuf, [jnp.maximum(iota - 1, 0)])
            cprev = plsc.load_gather(cbuf, [jnp.maximum(iota - 1, 0)])
            mend = (sk != knext) | (iota == 15)
            mst = (sk != kprev) & (iota > 0)
            plsc.addupdate_scatter(hist, [sk], cs, mask=mend)
            plsc.addupdate_scatter(hist, [sk], -cprev, mask=mst)

        def edge_body(j, _):
            ks = edge_v[1, pl.ds(j * 16, 16)]
            rw = edge_v[0, pl.ds(j * 16, 16)]
            s = val_v[pl.ds(j * 16, 16)]
            e = jnp.exp(s - mv) * ziv
            accum(degh, ks, e)
            vw = jnp.where(ks == 0, e, 0.0)

            @pl.when(jnp.max(vw) > 0.0)
            def _():
                accum(wh, rw, vw)

            return 0

        lax.fori_loop(0, chunks, edge_body, 0)
        pltpu.sync_copy(degh, deg_out.at[pl.ds(wid * NP, NP)])
        pltpu.sync_copy(wh, w_out.at[pl.ds(wid * NP, NP)])

    return sck(edge2d, scores2d, mrow, zirow)


def _stage_e(deg_part, w_part, enc, W_gcn, b_gcn2, h2, phx, pv, W_comb,
             b_comb2, W_out, b_out2, W_wh, N, V, EXT, NP):
    def body(degp_ref, wp_ref, enc_ref, wgcn_ref, bgcn_ref, h2_ref, phx_ref,
             pv_ref, wcomb_ref, bcomb_ref, wout_ref, bout_ref, wwh_ref,
             main_ref, ext_ref):
        deg_acc = degp_ref[pl.ds(0, NP)]
        w_acc = wp_ref[pl.ds(0, NP)]
        for t in range(1, _NW):
            deg_acc = deg_acc + degp_ref[pl.ds(t * NP, NP)]
            w_acc = w_acc + wp_ref[pl.ds(t * NP, NP)]
        deg = deg_acc.reshape(1, NP) + 1.0
        dinv = lax.rsqrt(deg)
        w = w_acc.reshape(1, NP)
        u = dinv * w
        col = lax.broadcasted_iota(jnp.int32, (1, deg.shape[1]), 1)
        u = u + jnp.where(col == 0, dinv, 0.0)
        u = u[:, :enc_ref.shape[0]]
        v = lax.dot_general(u, enc_ref[...], (((1,), (0,)), ((), ())),
                            preferred_element_type=jnp.float32)
        aa = lax.dot_general(v, wgcn_ref[...], (((1,), (1,)), ((), ())),
                             preferred_element_type=jnp.float32)
        dinv0 = dinv[:, 0:1]
        attn_applied = dinv0 * aa + bgcn_ref[...]
        p = jnp.sum(attn_applied * wwh_ref[...]) + phx_ref[0, 0]
        p_gen = jax.nn.sigmoid(p) + 1e-07
        atten_p = pv_ref[...] * (1.0 - p_gen + 1e-07)
        cat = jnp.concatenate([h2_ref[...], attn_applied], axis=1)
        comb = jnp.tanh(
            lax.dot_general(cat, wcomb_ref[...], (((1,), (1,)), ((), ())),
                            preferred_element_type=jnp.float32)
            + bcomb_ref[...])
        logits = lax.dot_general(comb, wout_ref[...], (((1,), (1,)), ((), ())),
                                 preferred_element_type=jnp.float32) \
            + bout_ref[...]
        mx = jnp.max(logits, axis=1, keepdims=True)
        ex = jnp.exp(logits - mx)
        ssum = jnp.sum(ex, axis=1, keepdims=True)
        main_ref[...] = jnp.log(ex / ssum * p_gen)
        ext_ref[...] = jnp.log(atten_p)

    return pl.pallas_call(
        body,
        out_shape=[
            jax.ShapeDtypeStruct((1, V), jnp.float32),
            jax.ShapeDtypeStruct((1, EXT), jnp.float32),
        ],
    )(deg_part, w_part, enc, W_gcn, b_gcn2, h2, phx, pv, W_comb, b_comb2,
      W_out, b_out2, W_wh)


def kernel(input, hidden_h, hidden_c, encoder_outputs, syn_embeddeds,
           edge_index, pg_mat, emb_table, W_attn, W_gcn, b_gcn, W_comb,
           b_comb, W_ih, W_hh, b_ih, b_hh, W_out, b_out, W_wh, W_ws, W_wx,
           b_wx):
    N, _ = encoder_outputs.shape
    E, _ = syn_embeddeds.shape
    EXT = pg_mat.shape[1]
    V = W_out.shape[0]

    h0 = hidden_h.reshape(1, H)
    c0 = hidden_c.reshape(1, H)
    emb_row = lax.dynamic_slice(emb_table, (input.astype(jnp.int32)[0], 0),
                                (1, H))
    h2, c2, q, phx, h23, c23 = _stage_a(
        emb_row, h0, c0, W_ih, W_hh,
        b_ih.reshape(1, HID4), b_hh.reshape(1, HID4), W_attn,
        W_ws, W_wx, b_wx.reshape(1, 1))

    scores, mrow, zirow = _stage_b(q, syn_embeddeds, E, 16000)
    aw, pv = _stage_c(scores, mrow, zirow, pg_mat, E, EXT, 3200)

    deg_part, w_part = _sc_partials(edge_index.astype(jnp.int32), scores,
                                    mrow, zirow, E, N)
    NP = deg_part.shape[0] // _NW

    out_main, out_ext = _stage_e(
        deg_part, w_part, encoder_outputs,
        W_gcn, b_gcn.reshape(1, H), h2, phx, pv, W_comb,
        b_comb.reshape(1, H), W_out, b_out.reshape(1, V), W_wh, N, V, EXT,
        NP)

    out = jnp.concatenate([out_main, out_ext], axis=1)
    return (out, h23, c23, aw)


# speedup vs baseline: 23.5810x; 1.0100x over previous
import functools

import jax
import jax.numpy as jnp
from jax import lax
from jax.experimental import pallas as pl
from jax.experimental.pallas import tpu as pltpu
from jax.experimental.pallas import tpu_sc as plsc

H = 256
HID4 = 4 * H

_NC = 2
_NS = 16
_NW = _NC * _NS


def _stage_a(emb_row, h0, c0, W_ih, W_hh, b_ih2, b_hh2, W_attn,
             W_ws, W_wx, b_wx2):
    def body(emb_ref, h_ref, c_ref, wih_ref, whh_ref, bih_ref,
             bhh_ref, wattn_ref, wws_ref, wwx_ref, bwx_ref,
             h2_ref, c2_ref, q_ref, phx_ref, h23_ref, c23_ref):
        x = emb_ref[...]
        h = h_ref[...]
        c = c_ref[...]
        g = (lax.dot_general(x, wih_ref[...], (((1,), (1,)), ((), ())),
                             preferred_element_type=jnp.float32)
             + lax.dot_general(h, whh_ref[...], (((1,), (1,)), ((), ())),
                               preferred_element_type=jnp.float32)
             + bih_ref[...] + bhh_ref[...])
        i_g = jax.nn.sigmoid(g[:, 0:H])
        f_g = jax.nn.sigmoid(g[:, H:2 * H])
        g_g = jnp.tanh(g[:, 2 * H:3 * H])
        o_g = jax.nn.sigmoid(g[:, 3 * H:4 * H])
        c2 = f_g * c + i_g * g_g
        h2 = o_g * jnp.tanh(c2)
        h2_ref[...] = h2
        c2_ref[...] = c2
        q_ref[...] = lax.dot_general(h2, wattn_ref[...],
                                     (((1,), (1,)), ((), ())),
                                     preferred_element_type=jnp.float32)
        phx = (jnp.sum(h2 * wws_ref[...]) + jnp.sum(x * wwx_ref[...])
               + bwx_ref[0, 0])
        phx_ref[...] = jnp.full((1, 1), 0.0, jnp.float32) + phx
        h23_ref[...] = h2.reshape(1, 1, H)
        c23_ref[...] = c2.reshape(1, 1, H)

    return pl.pallas_call(
        body,
        out_shape=[
            jax.ShapeDtypeStruct((1, H), jnp.float32),
            jax.ShapeDtypeStruct((1, H), jnp.float32),
            jax.ShapeDtypeStruct((1, H), jnp.float32),
            jax.ShapeDtypeStruct((1, 1), jnp.float32),
            jax.ShapeDtypeStruct((1, 1, H), jnp.float32),
            jax.ShapeDtypeStruct((1, 1, H), jnp.float32),
        ],
    )(emb_row, h0, c0, W_ih, W_hh, b_ih2, b_hh2, W_attn, W_ws,
      W_wx, b_wx2)


def _stage_b(q, syn, E, BE):
    steps = E // BE

    def body(q_ref, syn_ref, s_ref, m_ref, zi_ref, scr):
        i = pl.program_id(0)

        @pl.when(i == 0)
        def _():
            scr[0] = -jnp.inf
            scr[1] = 0.0

        s = lax.dot_general(q_ref[...], syn_ref[...],
                            (((1,), (1,)), ((), ())),
                            preferred_element_type=jnp.float32)
        s_ref[...] = s
        m_old = scr[0]
        z_old = scr[1]
        m_blk = jnp.max(s)
        m_new = jnp.maximum(m_old, m_blk)
        z_new = z_old * jnp.exp(m_old - m_new) + jnp.sum(jnp.exp(s - m_new))
        scr[0] = m_new
        scr[1] = z_new

        @pl.when(i == steps - 1)
        def _():
            m_ref[...] = jnp.full((1, 128), m_new, jnp.float32)
            zi_ref[...] = jnp.full((1, 128), 1.0 / z_new, jnp.float32)

    return pl.pallas_call(
        body,
        grid=(steps,),
        in_specs=[
            pl.BlockSpec((1, H), lambda i: (0, 0)),
            pl.BlockSpec((BE, H), lambda i: (i, 0)),
        ],
        out_specs=[
            pl.BlockSpec((1, BE), lambda i: (0, i)),
            pl.BlockSpec((1, 128), lambda i: (0, 0)),
            pl.BlockSpec((1, 128), lambda i: (0, 0)),
        ],
        out_shape=[
            jax.ShapeDtypeStruct((1, E), jnp.float32),
            jax.ShapeDtypeStruct((1, 128), jnp.float32),
            jax.ShapeDtypeStruct((1, 128), jnp.float32),
        ],
        scratch_shapes=[pltpu.SMEM((2,), jnp.float32)],
    )(q, syn)


def _stage_c(scores, mrow, zirow, pg_mat, E, EXT, BC):
    steps = E // BC

    def body(s_ref, m_ref, zi_ref, pg_ref, aw_ref, pv_ref):
        i = pl.program_id(0)
        aw = jnp.exp(s_ref[...] - m_ref[0, 0]) * zi_ref[0, 0]
        aw_ref[...] = aw
        part = lax.dot_general(aw, pg_ref[...], (((1,), (0,)), ((), ())),
                               preferred_element_type=jnp.float32)

        @pl.when(i == 0)
        def _():
            pv_ref[...] = jnp.zeros_like(pv_ref)

        pv_ref[...] += part

    return pl.pallas_call(
        body,
        grid=(steps,),
        in_specs=[
            pl.BlockSpec((1, BC), lambda i: (0, i)),
            pl.BlockSpec((1, 128), lambda i: (0, 0)),
            pl.BlockSpec((1, 128), lambda i: (0, 0)),
            pl.BlockSpec((BC, EXT), lambda i: (i, 0)),
        ],
        out_specs=[
            pl.BlockSpec((1, BC), lambda i: (0, i)),
            pl.BlockSpec((1, EXT), lambda i: (0, 0)),
        ],
        out_shape=[
            jax.ShapeDtypeStruct((1, E), jnp.float32),
            jax.ShapeDtypeStruct((1, EXT), jnp.float32),
        ],
    )(scores, mrow, zirow, pg_mat)


def _sc_partials(edge2d, scores2d, mrow, zirow, E, N):
    grains = E // 128
    base_len = (grains // _NW) * 128
    last_len = E - base_len * (_NW - 1)
    pad = last_len
    chunks = pad // 16
    NP = ((N + 127) // 128) * 128
    mesh = plsc.VectorSubcoreMesh(core_axis_name="c", subcore_axis_name="s",
                                  num_cores=_NC, num_subcores=_NS)

    @functools.partial(
        pl.kernel,
        out_type=(jax.ShapeDtypeStruct((_NW * NP,), jnp.float32),
                  jax.ShapeDtypeStruct((_NW * NP,), jnp.float32)),
        mesh=mesh,
        compiler_params=pltpu.CompilerParams(needs_layout_passes=False),
        scratch_types=[
            pltpu.VMEM((2, pad), jnp.int32),
            pltpu.VMEM((pad,), jnp.float32),
            pltpu.VMEM((16,), jnp.int32),
            pltpu.VMEM((16,), jnp.float32),
            pltpu.VMEM((NP,), jnp.float32),
            pltpu.VMEM((NP,), jnp.float32),
            pltpu.VMEM((256,), jnp.float32),
        ],
    )
    def sck(edge_hbm, sc_hbm, m_hbm, zi_hbm, deg_out, w_out,
            edge_v, val_v, kbuf, cbuf, degh, wh, stat_v):
        wid = lax.axis_index("s") * _NC + lax.axis_index("c")
        base = wid * base_len
        iota = lax.iota(jnp.int32, 16)
        zf = jnp.zeros((16,), jnp.float32)
        zi_ = jnp.zeros((16,), jnp.int32)
        neg = jnp.full((16,), -1e30, jnp.float32)

        def fill_body(j, _):
            off = base_len + j * 16
            edge_v[0, pl.ds(off, 16)] = zi_
            edge_v[1, pl.ds(off, 16)] = zi_
            val_v[pl.ds(off, 16)] = neg
            return 0

        lax.fori_loop(0, (pad - base_len) // 16, fill_body, 0)

        @pl.when(wid == _NW - 1)
        def _():
            pltpu.sync_copy(edge_hbm.at[pl.ds(0, 2), pl.ds(base, last_len)],
                            edge_v.at[pl.ds(0, 2), pl.ds(0, last_len)])
            pltpu.sync_copy(sc_hbm.at[0, pl.ds(base, last_len)],
                            val_v.at[pl.ds(0, last_len)])

        @pl.when(wid < _NW - 1)
        def _():
            pltpu.sync_copy(edge_hbm.at[pl.ds(0, 2), pl.ds(base, base_len)],
                            edge_v.at[pl.ds(0, 2), pl.ds(0, base_len)])
            pltpu.sync_copy(sc_hbm.at[0, pl.ds(base, base_len)],
                            val_v.at[pl.ds(0, base_len)])

        pltpu.sync_copy(m_hbm.at[0, pl.ds(0, 128)], stat_v.at[pl.ds(0, 128)])
        pltpu.sync_copy(zi_hbm.at[0, pl.ds(0, 128)],
                        stat_v.at[pl.ds(128, 128)])

        def zero_body(j, _):
            degh[pl.ds(j * 16, 16)] = zf
            wh[pl.ds(j * 16, 16)] = zf
            return 0

        lax.fori_loop(0, NP // 16, zero_body, 0)
        mv = stat_v[pl.ds(0, 16)]
        ziv = stat_v[pl.ds(128, 16)]

        def accum(hist, keys, vals):
            sk, sv = plsc.sort_key_val(keys, vals)
            cs = plsc.cumsum(sv)
            kbuf[...] = sk
            cbuf[...] = cs
            knext = plsc.load_gather(kbuf, [jnp.minimum(iota + 1, 15)])
            kprev = plsc.load_gather(kbuf, [jnp.maximum(iota - 1, 0)])
            cprev = plsc.load_gather(cbuf, [jnp.maximum(iota - 1, 0)])
            mend = (sk != knext) | (iota == 15)
            mst = (sk != kprev) & (iota > 0)
            plsc.addupdate_scatter(hist, [sk], cs, mask=mend)
            plsc.addupdate_scatter(hist, [sk], -cprev, mask=mst)

        def edge_body(j, _):
            ks = edge_v[1, pl.ds(j * 16, 16)]
            rw = edge_v[0, pl.ds(j * 16, 16)]
            s = val_v[pl.ds(j * 16, 16)]
            e = jnp.exp(s - mv) * ziv
            accum(degh, ks, e)
            vw = jnp.where(ks == 0, e, 0.0)

            @pl.when(jnp.max(vw) > 0.0)
            def _():
                accum(wh, rw, vw)

            return 0

        lax.fori_loop(0, chunks, edge_body, 0)
        pltpu.sync_copy(degh, deg_out.at[pl.ds(wid * NP, NP)])
        pltpu.sync_copy(wh, w_out.at[pl.ds(wid * NP, NP)])

    return sck(edge2d, scores2d, mrow, zirow)


def _stage_e(deg_part, w_part, enc, W_gcn, b_gcn2, h2, phx, pv, W_comb,
             b_comb2, W_out, b_out2, W_wh, N, V, EXT, NP):
    def body(degp_ref, wp_ref, enc_ref, wgcn_ref, bgcn_ref, h2_ref, phx_ref,
             pv_ref, wcomb_ref, bcomb_ref, wout_ref, bout_ref, wwh_ref,
             main_ref, ext_ref):
        deg_acc = degp_ref[pl.ds(0, NP)]
        w_acc = wp_ref[pl.ds(0, NP)]
        for t in range(1, _NW):
            deg_acc = deg_acc + degp_ref[pl.ds(t * NP, NP)]
            w_acc = w_acc + wp_ref[pl.ds(t * NP, NP)]
        deg = deg_acc.reshape(1, NP) + 1.0
        dinv = lax.rsqrt(deg)
        w = w_acc.reshape(1, NP)
        u = dinv * w
        col = lax.broadcasted_iota(jnp.int32, (1, deg.shape[1]), 1)
        u = u + jnp.where(col == 0, dinv, 0.0)
        u = u[:, :enc_ref.shape[0]]
        v = lax.dot_general(u, enc_ref[...], (((1,), (0,)), ((), ())),
                            preferred_element_type=jnp.float32)
        aa = lax.dot_general(v, wgcn_ref[...], (((1,), (1,)), ((), ())),
                             preferred_element_type=jnp.float32)
        dinv0 = dinv[:, 0:1]
        attn_applied = dinv0 * aa + bgcn_ref[...]
        p = jnp.sum(attn_applied * wwh_ref[...]) + phx_ref[0, 0]
        p_gen = jax.nn.sigmoid(p) + 1e-07
        atten_p = pv_ref[...] * (1.0 - p_gen + 1e-07)
        cat = jnp.concatenate([h2_ref[...], attn_applied], axis=1)
        comb = jnp.tanh(
            lax.dot_general(cat, wcomb_ref[...], (((1,), (1,)), ((), ())),
                            preferred_element_type=jnp.float32)
            + bcomb_ref[...])
        logits = lax.dot_general(comb, wout_ref[...], (((1,), (1,)), ((), ())),
                                 preferred_element_type=jnp.float32) \
            + bout_ref[...]
        mx = jnp.max(logits, axis=1, keepdims=True)
        ex = jnp.exp(logits - mx)
        ssum = jnp.sum(ex, axis=1, keepdims=True)
        main_ref[...] = jnp.log(ex / ssum * p_gen)
        ext_ref[...] = jnp.log(atten_p)

    return pl.pallas_call(
        body,
        out_shape=[
            jax.ShapeDtypeStruct((1, V), jnp.float32),
            jax.ShapeDtypeStruct((1, EXT), jnp.float32),
        ],
    )(deg_part, w_part, enc, W_gcn, b_gcn2, h2, phx, pv, W_comb, b_comb2,
      W_out, b_out2, W_wh)


def kernel(input, hidden_h, hidden_c, encoder_outputs, syn_embeddeds,
           edge_index, pg_mat, emb_table, W_attn, W_gcn, b_gcn, W_comb,
           b_comb, W_ih, W_hh, b_ih, b_hh, W_out, b_out, W_wh, W_ws, W_wx,
           b_wx):
    N, _ = encoder_outputs.shape
    E, _ = syn_embeddeds.shape
    EXT = pg_mat.shape[1]
    V = W_out.shape[0]

    h0 = hidden_h.reshape(1, H)
    c0 = hidden_c.reshape(1, H)
    emb_row = lax.dynamic_slice(emb_table, (input.astype(jnp.int32)[0], 0),
                                (1, H))
    h2, c2, q, phx, h23, c23 = _stage_a(
        emb_row, h0, c0, W_ih, W_hh,
        b_ih.reshape(1, HID4), b_hh.reshape(1, HID4), W_attn,
        W_ws, W_wx, b_wx.reshape(1, 1))

    scores, mrow, zirow = _stage_b(q, syn_embeddeds, E, 16000)
    aw, pv = _stage_c(scores, mrow, zirow, pg_mat, E, EXT, 6400)

    deg_part, w_part = _sc_partials(edge_index.astype(jnp.int32), scores,
                                    mrow, zirow, E, N)
    NP = deg_part.shape[0] // _NW

    out_main, out_ext = _stage_e(
        deg_part, w_part, encoder_outputs,
        W_gcn, b_gcn.reshape(1, H), h2, phx, pv, W_comb,
        b_comb.reshape(1, H), W_out, b_out.reshape(1, V), W_wh, N, V, EXT,
        NP)

    out = jnp.concatenate([out_main, out_ext], axis=1)
    return (out, h23, c23, aw)


# speedup vs baseline: 23.9209x; 1.0144x over previous
import functools

import jax
import jax.numpy as jnp
from jax import lax
from jax.experimental import pallas as pl
from jax.experimental.pallas import tpu as pltpu
from jax.experimental.pallas import tpu_sc as plsc

H = 256
HID4 = 4 * H

_NC = 2
_NS = 16
_NW = _NC * _NS


def _stage_ab(emb_row, h0, c0, W_ih, W_hh, b_ih2, b_hh2, W_attn,
              W_ws, W_wx, b_wx2, syn, E, BE):
    steps = E // BE

    def body(emb_ref, h_ref, c_ref, wih_ref, whh_ref, bih_ref,
             bhh_ref, wattn_ref, wws_ref, wwx_ref, bwx_ref, syn_ref,
             s_ref, m_ref, zi_ref, h2_ref, phx_ref, h23_ref, c23_ref,
             q_scr, scr):
        i = pl.program_id(0)

        @pl.when(i == 0)
        def _():
            x = emb_ref[...]
            h = h_ref[...]
            c = c_ref[...]
            g = (lax.dot_general(x, wih_ref[...], (((1,), (1,)), ((), ())),
                                 preferred_element_type=jnp.float32)
                 + lax.dot_general(h, whh_ref[...], (((1,), (1,)), ((), ())),
                                   preferred_element_type=jnp.float32)
                 + bih_ref[...] + bhh_ref[...])
            i_g = jax.nn.sigmoid(g[:, 0:H])
            f_g = jax.nn.sigmoid(g[:, H:2 * H])
            g_g = jnp.tanh(g[:, 2 * H:3 * H])
            o_g = jax.nn.sigmoid(g[:, 3 * H:4 * H])
            c2 = f_g * c + i_g * g_g
            h2 = o_g * jnp.tanh(c2)
            h2_ref[...] = h2
            q_scr[...] = lax.dot_general(h2, wattn_ref[...],
                                         (((1,), (1,)), ((), ())),
                                         preferred_element_type=jnp.float32)
            phx = (jnp.sum(h2 * wws_ref[...]) + jnp.sum(x * wwx_ref[...])
                   + bwx_ref[0, 0])
            phx_ref[...] = jnp.full((1, 1), 0.0, jnp.float32) + phx
            h23_ref[...] = h2.reshape(1, 1, H)
            c23_ref[...] = c2.reshape(1, 1, H)
            scr[0] = -jnp.inf
            scr[1] = 0.0

        s = lax.dot_general(q_scr[...], syn_ref[...],
                            (((1,), (1,)), ((), ())),
                            preferred_element_type=jnp.float32)
        s_ref[...] = s
        m_old = scr[0]
        z_old = scr[1]
        m_blk = jnp.max(s)
        m_new = jnp.maximum(m_old, m_blk)
        z_new = z_old * jnp.exp(m_old - m_new) + jnp.sum(jnp.exp(s - m_new))
        scr[0] = m_new
        scr[1] = z_new

        @pl.when(i == steps - 1)
        def _():
            m_ref[...] = jnp.full((1, 128), m_new, jnp.float32)
            zi_ref[...] = jnp.full((1, 128), 1.0 / z_new, jnp.float32)

    zero = lambda i: (0, 0)
    zero3 = lambda i: (0, 0, 0)
    return pl.pallas_call(
        body,
        grid=(steps,),
        in_specs=[
            pl.BlockSpec((1, H), zero),
            pl.BlockSpec((1, H), zero),
            pl.BlockSpec((1, H), zero),
            pl.BlockSpec((HID4, H), zero),
            pl.BlockSpec((HID4, H), zero),
            pl.BlockSpec((1, HID4), zero),
            pl.BlockSpec((1, HID4), zero),
            pl.BlockSpec((H, H), zero),
            pl.BlockSpec((1, H), zero),
            pl.BlockSpec((1, H), zero),
            pl.BlockSpec((1, 1), zero),
            pl.BlockSpec((BE, H), lambda i: (i, 0)),
        ],
        out_specs=[
            pl.BlockSpec((1, BE), lambda i: (0, i)),
            pl.BlockSpec((1, 128), zero),
            pl.BlockSpec((1, 128), zero),
            pl.BlockSpec((1, H), zero),
            pl.BlockSpec((1, 1), zero),
            pl.BlockSpec((1, 1, H), zero3),
            pl.BlockSpec((1, 1, H), zero3),
        ],
        out_shape=[
            jax.ShapeDtypeStruct((1, E), jnp.float32),
            jax.ShapeDtypeStruct((1, 128), jnp.float32),
            jax.ShapeDtypeStruct((1, 128), jnp.float32),
            jax.ShapeDtypeStruct((1, H), jnp.float32),
            jax.ShapeDtypeStruct((1, 1), jnp.float32),
            jax.ShapeDtypeStruct((1, 1, H), jnp.float32),
            jax.ShapeDtypeStruct((1, 1, H), jnp.float32),
        ],
        scratch_shapes=[pltpu.VMEM((1, H), jnp.float32),
                        pltpu.SMEM((2,), jnp.float32)],
    )(emb_row, h0, c0, W_ih, W_hh, b_ih2, b_hh2, W_attn, W_ws,
      W_wx, b_wx2, syn)


def _stage_c(scores, mrow, zirow, pg_mat, E, EXT, BC):
    steps = E // BC

    def body(s_ref, m_ref, zi_ref, pg_ref, aw_ref, pv_ref):
        i = pl.program_id(0)
        aw = jnp.exp(s_ref[...] - m_ref[0, 0]) * zi_ref[0, 0]
        aw_ref[...] = aw
        part = lax.dot_general(aw, pg_ref[...], (((1,), (0,)), ((), ())),
                               preferred_element_type=jnp.float32)

        @pl.when(i == 0)
        def _():
            pv_ref[...] = jnp.zeros_like(pv_ref)

        pv_ref[...] += part

    return pl.pallas_call(
        body,
        grid=(steps,),
        in_specs=[
            pl.BlockSpec((1, BC), lambda i: (0, i)),
            pl.BlockSpec((1, 128), lambda i: (0, 0)),
            pl.BlockSpec((1, 128), lambda i: (0, 0)),
            pl.BlockSpec((BC, EXT), lambda i: (i, 0)),
        ],
        out_specs=[
            pl.BlockSpec((1, BC), lambda i: (0, i)),
            pl.BlockSpec((1, EXT), lambda i: (0, 0)),
        ],
        out_shape=[
            jax.ShapeDtypeStruct((1, E), jnp.float32),
            jax.ShapeDtypeStruct((1, EXT), jnp.float32),
        ],
    )(scores, mrow, zirow, pg_mat)


def _sc_partials(edge2d, scores2d, mrow, zirow, E, N):
    grains = E // 128
    base_len = (grains // _NW) * 128
    last_len = E - base_len * (_NW - 1)
    pad = last_len
    chunks = pad // 16
    NP = ((N + 127) // 128) * 128
    mesh = plsc.VectorSubcoreMesh(core_axis_name="c", subcore_axis_name="s",
                                  num_cores=_NC, num_subcores=_NS)

    @functools.partial(
        pl.kernel,
        out_type=(jax.ShapeDtypeStruct((_NW * NP,), jnp.float32),
                  jax.ShapeDtypeStruct((_NW * NP,), jnp.float32)),
        mesh=mesh,
        compiler_params=pltpu.CompilerParams(needs_layout_passes=False),
        scratch_types=[
            pltpu.VMEM((2, pad), jnp.int32),
            pltpu.VMEM((pad,), jnp.float32),
            pltpu.VMEM((16,), jnp.int32),
            pltpu.VMEM((16,), jnp.float32),
            pltpu.VMEM((NP,), jnp.float32),
            pltpu.VMEM((NP,), jnp.float32),
            pltpu.VMEM((256,), jnp.float32),
        ],
    )
    def sck(edge_hbm, sc_hbm, m_hbm, zi_hbm, deg_out, w_out,
            edge_v, val_v, kbuf, cbuf, degh, wh, stat_v):
        wid = lax.axis_index("s") * _NC + lax.axis_index("c")
        base = wid * base_len
        iota = lax.iota(jnp.int32, 16)
        zf = jnp.zeros((16,), jnp.float32)
        zi_ = jnp.zeros((16,), jnp.int32)
        neg = jnp.full((16,), -1e30, jnp.float32)

        def fill_body(j, _):
            off = base_len + j * 16
            edge_v[0, pl.ds(off, 16)] = zi_
            edge_v[1, pl.ds(off, 16)] = zi_
            val_v[pl.ds(off, 16)] = neg
            return 0

        lax.fori_loop(0, (pad - base_len) // 16, fill_body, 0)

        @pl.when(wid == _NW - 1)
        def _():
            pltpu.sync_copy(edge_hbm.at[pl.ds(0, 2), pl.ds(base, last_len)],
                            edge_v.at[pl.ds(0, 2), pl.ds(0, last_len)])
            pltpu.sync_copy(sc_hbm.at[0, pl.ds(base, last_len)],
                            val_v.at[pl.ds(0, last_len)])

        @pl.when(wid < _NW - 1)
        def _():
            pltpu.sync_copy(edge_hbm.at[pl.ds(0, 2), pl.ds(base, base_len)],
                            edge_v.at[pl.ds(0, 2), pl.ds(0, base_len)])
            pltpu.sync_copy(sc_hbm.at[0, pl.ds(base, base_len)],
                            val_v.at[pl.ds(0, base_len)])

        pltpu.sync_copy(m_hbm.at[0, pl.ds(0, 128)], stat_v.at[pl.ds(0, 128)])
        pltpu.sync_copy(zi_hbm.at[0, pl.ds(0, 128)],
                        stat_v.at[pl.ds(128, 128)])

        def zero_body(j, _):
            degh[pl.ds(j * 16, 16)] = zf
            wh[pl.ds(j * 16, 16)] = zf
            return 0

        lax.fori_loop(0, NP // 16, zero_body, 0)
        mv = stat_v[pl.ds(0, 16)]
        ziv = stat_v[pl.ds(128, 16)]

        def accum(hist, keys, vals):
            sk, sv = plsc.sort_key_val(keys, vals)
            cs = plsc.cumsum(sv)
            kbuf[...] = sk
            cbuf[...] = cs
            knext = plsc.load_gather(kbuf, [jnp.minimum(iota + 1, 15)])
            kprev = plsc.load_gather(kbuf, [jnp.maximum(iota - 1, 0)])
            cprev = plsc.load_gather(cbuf, [jnp.maximum(iota - 1, 0)])
            mend = (sk != knext) | (iota == 15)
            mst = (sk != kprev) & (iota > 0)
            plsc.addupdate_scatter(hist, [sk], cs, mask=mend)
            plsc.addupdate_scatter(hist, [sk], -cprev, mask=mst)

        def edge_body(j, _):
            ks = edge_v[1, pl.ds(j * 16, 16)]
            rw = edge_v[0, pl.ds(j * 16, 16)]
            s = val_v[pl.ds(j * 16, 16)]
            e = jnp.exp(s - mv) * ziv
            accum(degh, ks, e)
            vw = jnp.where(ks == 0, e, 0.0)

            @pl.when(jnp.max(vw) > 0.0)
            def _():
                accum(wh, rw, vw)

            return 0

        lax.fori_loop(0, chunks, edge_body, 0)
        pltpu.sync_copy(degh, deg_out.at[pl.ds(wid * NP, NP)])
        pltpu.sync_copy(wh, w_out.at[pl.ds(wid * NP, NP)])

    return sck(edge2d, scores2d, mrow, zirow)


def _stage_e(deg_part, w_part, enc, W_gcn, b_gcn2, h2, phx, pv, W_comb,
             b_comb2, W_out, b_out2, W_wh, N, V, EXT, NP):
    def body(degp_ref, wp_ref, enc_ref, wgcn_ref, bgcn_ref, h2_ref, phx_ref,
             pv_ref, wcomb_ref, bcomb_ref, wout_ref, bout_ref, wwh_ref,
             main_ref, ext_ref):
        deg_acc = degp_ref[pl.ds(0, NP)]
        w_acc = wp_ref[pl.ds(0, NP)]
        for t in range(1, _NW):
            deg_acc = deg_acc + degp_ref[pl.ds(t * NP, NP)]
            w_acc = w_acc + wp_ref[pl.ds(t * NP, NP)]
        deg = deg_acc.reshape(1, NP) + 1.0
        dinv = lax.rsqrt(deg)
        w = w_acc.reshape(1, NP)
        u = dinv * w
        col = lax.broadcasted_iota(jnp.int32, (1, deg.shape[1]), 1)
        u = u + jnp.where(col == 0, dinv, 0.0)
        u = u[:, :enc_ref.shape[0]]
        v = lax.dot_general(u, enc_ref[...], (((1,), (0,)), ((), ())),
                            preferred_element_type=jnp.float32)
        aa = lax.dot_general(v, wgcn_ref[...], (((1,), (1,)), ((), ())),
                             preferred_element_type=jnp.float32)
        dinv0 = dinv[:, 0:1]
        attn_applied = dinv0 * aa + bgcn_ref[...]
        p = jnp.sum(attn_applied * wwh_ref[...]) + phx_ref[0, 0]
        p_gen = jax.nn.sigmoid(p) + 1e-07
        atten_p = pv_ref[...] * (1.0 - p_gen + 1e-07)
        cat = jnp.concatenate([h2_ref[...], attn_applied], axis=1)
        comb = jnp.tanh(
            lax.dot_general(cat, wcomb_ref[...], (((1,), (1,)), ((), ())),
                            preferred_element_type=jnp.float32)
            + bcomb_ref[...])
        logits = lax.dot_general(comb, wout_ref[...], (((1,), (1,)), ((), ())),
                                 preferred_element_type=jnp.float32) \
            + bout_ref[...]
        mx = jnp.max(logits, axis=1, keepdims=True)
        ex = jnp.exp(logits - mx)
        ssum = jnp.sum(ex, axis=1, keepdims=True)
        main_ref[...] = jnp.log(ex / ssum * p_gen)
        ext_ref[...] = jnp.log(atten_p)

    return pl.pallas_call(
        body,
        out_shape=[
            jax.ShapeDtypeStruct((1, V), jnp.float32),
            jax.ShapeDtypeStruct((1, EXT), jnp.float32),
        ],
    )(deg_part, w_part, enc, W_gcn, b_gcn2, h2, phx, pv, W_comb, b_comb2,
      W_out, b_out2, W_wh)


def kernel(input, hidden_h, hidden_c, encoder_outputs, syn_embeddeds,
           edge_index, pg_mat, emb_table, W_attn, W_gcn, b_gcn, W_comb,
           b_comb, W_ih, W_hh, b_ih, b_hh, W_out, b_out, W_wh, W_ws, W_wx,
           b_wx):
    N, _ = encoder_outputs.shape
    E, _ = syn_embeddeds.shape
    EXT = pg_mat.shape[1]
    V = W_out.shape[0]

    h0 = hidden_h.reshape(1, H)
    c0 = hidden_c.reshape(1, H)
    emb_row = lax.dynamic_slice(emb_table, (input.astype(jnp.int32)[0], 0),
                                (1, H))
    scores, mrow, zirow, h2, phx, h23, c23 = _stage_ab(
        emb_row, h0, c0, W_ih, W_hh,
        b_ih.reshape(1, HID4), b_hh.reshape(1, HID4), W_attn,
        W_ws, W_wx, b_wx.reshape(1, 1), syn_embeddeds, E, 16000)
    aw, pv = _stage_c(scores, mrow, zirow, pg_mat, E, EXT, 6400)

    deg_part, w_part = _sc_partials(edge_index.astype(jnp.int32), scores,
                                    mrow, zirow, E, N)
    NP = deg_part.shape[0] // _NW

    out_main, out_ext = _stage_e(
        deg_part, w_part, encoder_outputs,
        W_gcn, b_gcn.reshape(1, H), h2, phx, pv, W_comb,
        b_comb.reshape(1, H), W_out, b_out.reshape(1, V), W_wh, N, V, EXT,
        NP)

    out = jnp.concatenate([out_main, out_ext], axis=1)
    return (out, h23, c23, aw)
